# initial kernel scaffold (unmeasured)
import jax
import jax.numpy as jnp
from jax import lax
from jax.experimental import pallas as pl
from jax.experimental.pallas import tpu as pltpu

N_DEV = 4
SQ = 2048
SKV_SHARD = 2048
SKV = N_DEV * SKV_SHARD
HSH = 8
DH = 128
DM = 1024
BAND = 128
NGLOB = 32
SCALE = 0.08838834764831843
BANDK = SKV_SHARD + 256


def _a2a_body(kt_ref, vt_ref, kf_ref, vf_ref, ksend, krecv, vsend, vrecv, csem):
    me = lax.axis_index("i")

    lk = pltpu.make_async_copy(
        kt_ref.at[pl.ds(me * HSH, HSH)], kf_ref.at[pl.ds(me * HSH, HSH)],
        csem.at[0])
    lv = pltpu.make_async_copy(
        vt_ref.at[pl.ds(me * HSH, HSH)], vf_ref.at[pl.ds(me * HSH, HSH)],
        csem.at[1])
    lk.start()
    lv.start()

    rdmas = []
    for d in (1, 2, 3):
        p = (me + d) % N_DEV
        rk = pltpu.make_async_remote_copy(
            src_ref=kt_ref.at[pl.ds(p * HSH, HSH)],
            dst_ref=kf_ref.at[pl.ds(me * HSH, HSH)],
            send_sem=ksend.at[d],
            recv_sem=krecv.at[d],
            device_id=(p,),
            device_id_type=pl.DeviceIdType.MESH,
        )
        rv = pltpu.make_async_remote_copy(
            src_ref=vt_ref.at[pl.ds(p * HSH, HSH)],
            dst_ref=vf_ref.at[pl.ds(me * HSH, HSH)],
            send_sem=vsend.at[d],
            recv_sem=vrecv.at[d],
            device_id=(p,),
            device_id_type=pl.DeviceIdType.MESH,
        )
        rk.start()
        rv.start()
        rdmas.append((rk, rv))

    for rk, rv in rdmas:
        rk.wait()
        rv.wait()
    lk.wait()
    lv.wait()


def _a2a(kt, vt):
    return pl.pallas_call(
        _a2a_body,
        out_shape=[
            jax.ShapeDtypeStruct((N_DEV * HSH, SKV_SHARD, DH), jnp.bfloat16),
            jax.ShapeDtypeStruct((N_DEV * HSH, SKV_SHARD, DH), jnp.bfloat16),
        ],
        in_specs=[
            pl.BlockSpec(memory_space=pltpu.ANY),
            pl.BlockSpec(memory_space=pltpu.ANY),
        ],
        out_specs=[
            pl.BlockSpec(memory_space=pltpu.ANY),
            pl.BlockSpec(memory_space=pltpu.ANY),
        ],
        scratch_shapes=[
            pltpu.SemaphoreType.DMA((N_DEV,)),
            pltpu.SemaphoreType.DMA((N_DEV,)),
            pltpu.SemaphoreType.DMA((N_DEV,)),
            pltpu.SemaphoreType.DMA((N_DEV,)),
            pltpu.SemaphoreType.DMA((2,)),
        ],
    )(kt, vt)


def _attn_body(q_ref, kf_ref, vf_ref, o_ref):
    for h in range(HSH):
        kb = jnp.concatenate(
            [kf_ref[0 * HSH + h], kf_ref[1 * HSH + h, :BANDK - SKV_SHARD]], axis=0)
        vb = jnp.concatenate(
            [vf_ref[0 * HSH + h], vf_ref[1 * HSH + h, :BANDK - SKV_SHARD]], axis=0)
        for qb in range(4):
            q = q_ref[h, qb * 512:(qb + 1) * 512, :]
            s = lax.dot_general(q, kb, (((1,), (1,)), ((), ())),
                                preferred_element_type=jnp.float32) * SCALE
            qi = qb * 512 + lax.broadcasted_iota(jnp.int32, (512, BANDK), 0)
            ki = lax.broadcasted_iota(jnp.int32, (512, BANDK), 1)
            mask = ((jnp.abs(qi - ki) <= BAND) | (ki < NGLOB)) & (qi >= NGLOB)
            s = jnp.where(mask, s, -1e9)
            m = jnp.max(s, axis=1, keepdims=True)
            w = jnp.exp(s - m)
            w = w / jnp.sum(w, axis=1, keepdims=True)
            ctx = lax.dot_general(w.astype(jnp.bfloat16), vb,
                                  (((1,), (0,)), ((), ())),
                                  preferred_element_type=jnp.float32)
            o_ref[h, qb * 512:(qb + 1) * 512, :] = ctx.astype(jnp.bfloat16)

        kall = jnp.concatenate([kf_ref[s * HSH + h] for s in range(N_DEV)], axis=0)
        vall = jnp.concatenate([vf_ref[s * HSH + h] for s in range(N_DEV)], axis=0)
        q32 = q_ref[h, :NGLOB, :]
        s2 = lax.dot_general(q32, kall, (((1,), (1,)), ((), ())),
                             preferred_element_type=jnp.float32) * SCALE
        m2 = jnp.max(s2, axis=1, keepdims=True)
        w2 = jnp.exp(s2 - m2)
        w2 = w2 / jnp.sum(w2, axis=1, keepdims=True)
        ctx2 = lax.dot_general(w2.astype(jnp.bfloat16), vall,
                               (((1,), (0,)), ((), ())),
                               preferred_element_type=jnp.float32)
        o_ref[h, :NGLOB, :] = ctx2.astype(jnp.bfloat16)


def _attn(qh, kf, vf):
    return pl.pallas_call(
        _attn_body,
        out_shape=jax.ShapeDtypeStruct((HSH, SQ, DH), jnp.bfloat16),
        in_specs=[
            pl.BlockSpec(memory_space=pltpu.VMEM),
            pl.BlockSpec(memory_space=pltpu.VMEM),
            pl.BlockSpec(memory_space=pltpu.VMEM),
        ],
        out_specs=pl.BlockSpec(memory_space=pltpu.VMEM),
    )(qh, kf, vf)


def _ar_body(p_ref, o_ref, rs_ref, s1send, s1recv, s2send, s2recv):
    me = lax.axis_index("i")
    C = SQ // N_DEV

    r1 = []
    for d in (1, 2, 3):
        p = (me + d) % N_DEV
        rd = pltpu.make_async_remote_copy(
            src_ref=p_ref.at[pl.ds(p * C, C)],
            dst_ref=rs_ref.at[pl.ds(d * C, C)],
            send_sem=s1send.at[d],
            recv_sem=s1recv.at[d],
            device_id=(p,),
            device_id_type=pl.DeviceIdType.MESH,
        )
        rd.start()
        r1.append(rd)

    acc = p_ref[pl.ds(me * C, C), :]
    for d, rd in zip((1, 2, 3), r1):
        rd.wait()
        acc = acc + rs_ref[d * C:(d + 1) * C, :]
    o_ref[pl.ds(me * C, C), :] = acc

    r2 = []
    for d in (1, 2, 3):
        p = (me + d) % N_DEV
        rd = pltpu.make_async_remote_copy(
            src_ref=o_ref.at[pl.ds(me * C, C)],
            dst_ref=o_ref.at[pl.ds(me * C, C)],
            send_sem=s2send.at[d],
            recv_sem=s2recv.at[d],
            device_id=(p,),
            device_id_type=pl.DeviceIdType.MESH,
        )
        rd.start()
        r2.append(rd)
    for rd in r2:
        rd.wait()


def _allreduce(partial):
    return pl.pallas_call(
        _ar_body,
        out_shape=jax.ShapeDtypeStruct((SQ, DM), jnp.float32),
        in_specs=[pl.BlockSpec(memory_space=pltpu.VMEM)],
        out_specs=pl.BlockSpec(memory_space=pltpu.VMEM),
        scratch_shapes=[
            pltpu.VMEM((SQ, DM), jnp.float32),
            pltpu.SemaphoreType.DMA((N_DEV,)),
            pltpu.SemaphoreType.DMA((N_DEV,)),
            pltpu.SemaphoreType.DMA((N_DEV,)),
            pltpu.SemaphoreType.DMA((N_DEV,)),
        ],
    )(partial)


def kernel(x, Wq, K_ext, V_ext, Wo):
    xb = x[0].astype(jnp.bfloat16)
    q = xb @ Wq.astype(jnp.bfloat16)
    qh = q.reshape(SQ, HSH, DH).transpose(1, 0, 2)

    kt = K_ext[0].astype(jnp.bfloat16).transpose(1, 0, 2)
    vt = V_ext[0].astype(jnp.bfloat16).transpose(1, 0, 2)

    kf, vf = _a2a(kt, vt)
    ctx = _attn(qh, kf, vf)

    ctx2 = ctx.transpose(1, 0, 2).reshape(SQ, HSH * DH)
    partial = lax.dot_general(ctx2, Wo.astype(jnp.bfloat16),
                              (((1,), (0,)), ((), ())),
                              preferred_element_type=jnp.float32)
    out = _allreduce(partial)
    return out.reshape(1, SQ, DM)


# baseline (device time: 419698 ns/iter reference)
import jax
import jax.numpy as jnp
from jax import lax
from jax.experimental import pallas as pl
from jax.experimental.pallas import tpu as pltpu

N_DEV = 4
SQ = 2048
SKV_SHARD = 2048
SKV = N_DEV * SKV_SHARD
HSH = 8
DH = 128
DM = 1024
BAND = 128
NGLOB = 32
SCALE = 0.08838834764831843
BANDK = SKV_SHARD + 256


def _a2a_body(kt_ref, vt_ref, kf_ref, vf_ref, ksend, krecv, vsend, vrecv, csem):
    me = lax.axis_index("i")

    lk = pltpu.make_async_copy(
        kt_ref.at[pl.ds(me * HSH, HSH)], kf_ref.at[pl.ds(me * HSH, HSH)],
        csem.at[0])
    lv = pltpu.make_async_copy(
        vt_ref.at[pl.ds(me * HSH, HSH)], vf_ref.at[pl.ds(me * HSH, HSH)],
        csem.at[1])
    lk.start()
    lv.start()

    rdmas = []
    for d in (1, 2, 3):
        p = (me + d) % N_DEV
        rk = pltpu.make_async_remote_copy(
            src_ref=kt_ref.at[pl.ds(p * HSH, HSH)],
            dst_ref=kf_ref.at[pl.ds(me * HSH, HSH)],
            send_sem=ksend.at[d],
            recv_sem=krecv.at[d],
            device_id=(p,),
            device_id_type=pl.DeviceIdType.MESH,
        )
        rv = pltpu.make_async_remote_copy(
            src_ref=vt_ref.at[pl.ds(p * HSH, HSH)],
            dst_ref=vf_ref.at[pl.ds(me * HSH, HSH)],
            send_sem=vsend.at[d],
            recv_sem=vrecv.at[d],
            device_id=(p,),
            device_id_type=pl.DeviceIdType.MESH,
        )
        rk.start()
        rv.start()
        rdmas.append((rk, rv))

    for rk, rv in rdmas:
        rk.wait()
        rv.wait()
    lk.wait()
    lv.wait()


def _a2a(kt, vt):
    return pl.pallas_call(
        _a2a_body,
        out_shape=[
            jax.ShapeDtypeStruct((N_DEV * HSH, SKV_SHARD, DH), jnp.bfloat16),
            jax.ShapeDtypeStruct((N_DEV * HSH, SKV_SHARD, DH), jnp.bfloat16),
        ],
        in_specs=[
            pl.BlockSpec(memory_space=pl.ANY),
            pl.BlockSpec(memory_space=pl.ANY),
        ],
        out_specs=[
            pl.BlockSpec(memory_space=pl.ANY),
            pl.BlockSpec(memory_space=pl.ANY),
        ],
        scratch_shapes=[
            pltpu.SemaphoreType.DMA((N_DEV,)),
            pltpu.SemaphoreType.DMA((N_DEV,)),
            pltpu.SemaphoreType.DMA((N_DEV,)),
            pltpu.SemaphoreType.DMA((N_DEV,)),
            pltpu.SemaphoreType.DMA((2,)),
        ],
    )(kt, vt)


def _attn_body(q_ref, kf_ref, vf_ref, o_ref):
    kb = jnp.concatenate(
        [kf_ref[0, 0], kf_ref[1, 0, :BANDK - SKV_SHARD]], axis=0)
    vb = jnp.concatenate(
        [vf_ref[0, 0], vf_ref[1, 0, :BANDK - SKV_SHARD]], axis=0)
    for qb in range(4):
        q = q_ref[0, qb * 512:(qb + 1) * 512, :]
        s = lax.dot_general(q, kb, (((1,), (1,)), ((), ())),
                            preferred_element_type=jnp.float32) * SCALE
        qi = qb * 512 + lax.broadcasted_iota(jnp.int32, (512, BANDK), 0)
        ki = lax.broadcasted_iota(jnp.int32, (512, BANDK), 1)
        mask = ((jnp.abs(qi - ki) <= BAND) | (ki < NGLOB)) & (qi >= NGLOB)
        s = jnp.where(mask, s, -1e9)
        m = jnp.max(s, axis=1, keepdims=True)
        w = jnp.exp(s - m)
        w = w / jnp.sum(w, axis=1, keepdims=True)
        ctx = lax.dot_general(w.astype(jnp.bfloat16), vb,
                              (((1,), (0,)), ((), ())),
                              preferred_element_type=jnp.float32)
        o_ref[0, qb * 512:(qb + 1) * 512, :] = ctx.astype(jnp.bfloat16)

    kall = jnp.concatenate([kf_ref[s, 0] for s in range(N_DEV)], axis=0)
    vall = jnp.concatenate([vf_ref[s, 0] for s in range(N_DEV)], axis=0)
    q32 = q_ref[0, :NGLOB, :]
    s2 = lax.dot_general(q32, kall, (((1,), (1,)), ((), ())),
                         preferred_element_type=jnp.float32) * SCALE
    m2 = jnp.max(s2, axis=1, keepdims=True)
    w2 = jnp.exp(s2 - m2)
    w2 = w2 / jnp.sum(w2, axis=1, keepdims=True)
    ctx2 = lax.dot_general(w2.astype(jnp.bfloat16), vall,
                           (((1,), (0,)), ((), ())),
                           preferred_element_type=jnp.float32)
    o_ref[0, :NGLOB, :] = ctx2.astype(jnp.bfloat16)


def _attn(qh, kf, vf):
    kf4 = kf.reshape(N_DEV, HSH, SKV_SHARD, DH)
    vf4 = vf.reshape(N_DEV, HSH, SKV_SHARD, DH)
    return pl.pallas_call(
        _attn_body,
        grid=(HSH,),
        out_shape=jax.ShapeDtypeStruct((HSH, SQ, DH), jnp.bfloat16),
        in_specs=[
            pl.BlockSpec((1, SQ, DH), lambda h: (h, 0, 0)),
            pl.BlockSpec((N_DEV, 1, SKV_SHARD, DH), lambda h: (0, h, 0, 0)),
            pl.BlockSpec((N_DEV, 1, SKV_SHARD, DH), lambda h: (0, h, 0, 0)),
        ],
        out_specs=pl.BlockSpec((1, SQ, DH), lambda h: (h, 0, 0)),
    )(qh, kf4, vf4)


def _ar_body(p_ref, o_ref, rs_ref, s1send, s1recv, s2send, s2recv):
    me = lax.axis_index("i")
    C = SQ // N_DEV

    r1 = []
    for d in (1, 2, 3):
        p = (me + d) % N_DEV
        rd = pltpu.make_async_remote_copy(
            src_ref=p_ref.at[pl.ds(p * C, C)],
            dst_ref=rs_ref.at[pl.ds(d * C, C)],
            send_sem=s1send.at[d],
            recv_sem=s1recv.at[d],
            device_id=(p,),
            device_id_type=pl.DeviceIdType.MESH,
        )
        rd.start()
        r1.append(rd)

    acc = p_ref[pl.ds(me * C, C), :]
    for d, rd in zip((1, 2, 3), r1):
        rd.wait()
        acc = acc + rs_ref[d * C:(d + 1) * C, :]
    o_ref[pl.ds(me * C, C), :] = acc

    r2 = []
    for d in (1, 2, 3):
        p = (me + d) % N_DEV
        rd = pltpu.make_async_remote_copy(
            src_ref=o_ref.at[pl.ds(me * C, C)],
            dst_ref=o_ref.at[pl.ds(me * C, C)],
            send_sem=s2send.at[d],
            recv_sem=s2recv.at[d],
            device_id=(p,),
            device_id_type=pl.DeviceIdType.MESH,
        )
        rd.start()
        r2.append(rd)
    for rd in r2:
        rd.wait()


def _allreduce(partial):
    return pl.pallas_call(
        _ar_body,
        out_shape=jax.ShapeDtypeStruct((SQ, DM), jnp.float32),
        in_specs=[pl.BlockSpec(memory_space=pltpu.VMEM)],
        out_specs=pl.BlockSpec(memory_space=pltpu.VMEM),
        scratch_shapes=[
            pltpu.VMEM((SQ, DM), jnp.float32),
            pltpu.SemaphoreType.DMA((N_DEV,)),
            pltpu.SemaphoreType.DMA((N_DEV,)),
            pltpu.SemaphoreType.DMA((N_DEV,)),
            pltpu.SemaphoreType.DMA((N_DEV,)),
        ],
    )(partial)


def kernel(x, Wq, K_ext, V_ext, Wo):
    xb = x[0].astype(jnp.bfloat16)
    q = xb @ Wq.astype(jnp.bfloat16)
    qh = q.reshape(SQ, HSH, DH).transpose(1, 0, 2)

    kt = K_ext[0].astype(jnp.bfloat16).transpose(1, 0, 2)
    vt = V_ext[0].astype(jnp.bfloat16).transpose(1, 0, 2)

    kf, vf = _a2a(kt, vt)
    ctx = _attn(qh, kf, vf)

    ctx2 = ctx.transpose(1, 0, 2).reshape(SQ, HSH * DH)
    partial = lax.dot_general(ctx2, Wo.astype(jnp.bfloat16),
                              (((1,), (0,)), ((), ())),
                              preferred_element_type=jnp.float32)
    out = _allreduce(partial)
    return out.reshape(1, SQ, DM)


# device time: 346294 ns/iter; 1.2120x vs baseline; 1.2120x over previous
import jax
import jax.numpy as jnp
from jax import lax
from jax.experimental import pallas as pl
from jax.experimental.pallas import tpu as pltpu

N_DEV = 4
SQ = 2048
SKV_SHARD = 2048
SKV = N_DEV * SKV_SHARD
HSH = 8
DH = 128
DM = 1024
BAND = 128
NGLOB = 32
SCALE = 0.08838834764831843
BANDK = SKV_SHARD + 256


def _a2a_body(kt_ref, vt_ref, kf_ref, vf_ref, ksend, krecv, vsend, vrecv, csem):
    me = lax.axis_index("i")

    lk = pltpu.make_async_copy(
        kt_ref.at[pl.ds(me * HSH, HSH)], kf_ref.at[pl.ds(me * HSH, HSH)],
        csem.at[0])
    lv = pltpu.make_async_copy(
        vt_ref.at[pl.ds(me * HSH, HSH)], vf_ref.at[pl.ds(me * HSH, HSH)],
        csem.at[1])
    lk.start()
    lv.start()

    rdmas = []
    for d in (1, 2, 3):
        p = (me + d) % N_DEV
        rk = pltpu.make_async_remote_copy(
            src_ref=kt_ref.at[pl.ds(p * HSH, HSH)],
            dst_ref=kf_ref.at[pl.ds(me * HSH, HSH)],
            send_sem=ksend.at[d],
            recv_sem=krecv.at[d],
            device_id=(p,),
            device_id_type=pl.DeviceIdType.MESH,
        )
        rv = pltpu.make_async_remote_copy(
            src_ref=vt_ref.at[pl.ds(p * HSH, HSH)],
            dst_ref=vf_ref.at[pl.ds(me * HSH, HSH)],
            send_sem=vsend.at[d],
            recv_sem=vrecv.at[d],
            device_id=(p,),
            device_id_type=pl.DeviceIdType.MESH,
        )
        rk.start()
        rv.start()
        rdmas.append((rk, rv))

    for rk, rv in rdmas:
        rk.wait()
        rv.wait()
    lk.wait()
    lv.wait()


def _a2a(kt, vt):
    return pl.pallas_call(
        _a2a_body,
        out_shape=[
            jax.ShapeDtypeStruct((N_DEV * HSH, SKV_SHARD, DH), jnp.bfloat16),
            jax.ShapeDtypeStruct((N_DEV * HSH, SKV_SHARD, DH), jnp.bfloat16),
        ],
        in_specs=[
            pl.BlockSpec(memory_space=pl.ANY),
            pl.BlockSpec(memory_space=pl.ANY),
        ],
        out_specs=[
            pl.BlockSpec(memory_space=pl.ANY),
            pl.BlockSpec(memory_space=pl.ANY),
        ],
        scratch_shapes=[
            pltpu.SemaphoreType.DMA((N_DEV,)),
            pltpu.SemaphoreType.DMA((N_DEV,)),
            pltpu.SemaphoreType.DMA((N_DEV,)),
            pltpu.SemaphoreType.DMA((N_DEV,)),
            pltpu.SemaphoreType.DMA((2,)),
        ],
    )(kt, vt)


def _attn_body(q_ref, kf_ref, vf_ref, o_ref):
    kb = jnp.concatenate(
        [kf_ref[0, 0], kf_ref[1, 0, :BANDK - SKV_SHARD]], axis=0)
    vb = jnp.concatenate(
        [vf_ref[0, 0], vf_ref[1, 0, :BANDK - SKV_SHARD]], axis=0)
    def _scores(q, k, qi0, ki0, glob_only=False):
        s = lax.dot_general(q, k, (((1,), (1,)), ((), ())),
                            preferred_element_type=jnp.float32) * SCALE
        qi = qi0 + lax.broadcasted_iota(jnp.int32, s.shape, 0)
        ki = ki0 + lax.broadcasted_iota(jnp.int32, s.shape, 1)
        if glob_only:
            mask = ki < NGLOB
        else:
            mask = ((jnp.abs(qi - ki) <= BAND) | (ki < NGLOB)) & (qi >= NGLOB)
        return jnp.where(mask, s, -1e9)

    q = q_ref[0, :512, :]
    s = _scores(q, kb[:768], 0, 0)
    m = jnp.max(s, axis=1, keepdims=True)
    w = jnp.exp(s - m)
    w = w / jnp.sum(w, axis=1, keepdims=True)
    ctx = lax.dot_general(w.astype(jnp.bfloat16), vb[:768],
                          (((1,), (0,)), ((), ())),
                          preferred_element_type=jnp.float32)
    o_ref[0, :512, :] = ctx.astype(jnp.bfloat16)

    for qb in (1, 2, 3):
        lo = qb * 512 - BAND
        q = q_ref[0, qb * 512:(qb + 1) * 512, :]
        sg = _scores(q, kb[:128], qb * 512, 0, glob_only=True)
        sb = _scores(q, kb[lo:lo + 768], qb * 512, lo)
        m = jnp.maximum(jnp.max(sg, axis=1, keepdims=True),
                        jnp.max(sb, axis=1, keepdims=True))
        wg = jnp.exp(sg - m)
        wb = jnp.exp(sb - m)
        l = jnp.sum(wg, axis=1, keepdims=True) + jnp.sum(wb, axis=1, keepdims=True)
        ctx = (lax.dot_general(wg.astype(jnp.bfloat16), vb[:128],
                               (((1,), (0,)), ((), ())),
                               preferred_element_type=jnp.float32)
               + lax.dot_general(wb.astype(jnp.bfloat16), vb[lo:lo + 768],
                                 (((1,), (0,)), ((), ())),
                                 preferred_element_type=jnp.float32)) / l
        o_ref[0, qb * 512:(qb + 1) * 512, :] = ctx.astype(jnp.bfloat16)

    kall = jnp.concatenate([kf_ref[s, 0] for s in range(N_DEV)], axis=0)
    vall = jnp.concatenate([vf_ref[s, 0] for s in range(N_DEV)], axis=0)
    q32 = q_ref[0, :NGLOB, :]
    s2 = lax.dot_general(q32, kall, (((1,), (1,)), ((), ())),
                         preferred_element_type=jnp.float32) * SCALE
    m2 = jnp.max(s2, axis=1, keepdims=True)
    w2 = jnp.exp(s2 - m2)
    w2 = w2 / jnp.sum(w2, axis=1, keepdims=True)
    ctx2 = lax.dot_general(w2.astype(jnp.bfloat16), vall,
                           (((1,), (0,)), ((), ())),
                           preferred_element_type=jnp.float32)
    o_ref[0, :NGLOB, :] = ctx2.astype(jnp.bfloat16)


def _attn(qh, kf, vf):
    kf4 = kf.reshape(N_DEV, HSH, SKV_SHARD, DH)
    vf4 = vf.reshape(N_DEV, HSH, SKV_SHARD, DH)
    return pl.pallas_call(
        _attn_body,
        grid=(HSH,),
        out_shape=jax.ShapeDtypeStruct((HSH, SQ, DH), jnp.bfloat16),
        in_specs=[
            pl.BlockSpec((1, SQ, DH), lambda h: (h, 0, 0)),
            pl.BlockSpec((N_DEV, 1, SKV_SHARD, DH), lambda h: (0, h, 0, 0)),
            pl.BlockSpec((N_DEV, 1, SKV_SHARD, DH), lambda h: (0, h, 0, 0)),
        ],
        out_specs=pl.BlockSpec((1, SQ, DH), lambda h: (h, 0, 0)),
    )(qh, kf4, vf4)


def _ar_body(p_ref, o_ref, rs_ref, s1send, s1recv, s2send, s2recv):
    me = lax.axis_index("i")
    C = SQ // N_DEV

    r1 = []
    for d in (1, 2, 3):
        p = (me + d) % N_DEV
        rd = pltpu.make_async_remote_copy(
            src_ref=p_ref.at[pl.ds(p * C, C)],
            dst_ref=rs_ref.at[pl.ds(d * C, C)],
            send_sem=s1send.at[d],
            recv_sem=s1recv.at[d],
            device_id=(p,),
            device_id_type=pl.DeviceIdType.MESH,
        )
        rd.start()
        r1.append(rd)

    acc = p_ref[pl.ds(me * C, C), :].astype(jnp.float32)
    for d, rd in zip((1, 2, 3), r1):
        rd.wait()
        acc = acc + rs_ref[d * C:(d + 1) * C, :].astype(jnp.float32)
    o_ref[pl.ds(me * C, C), :] = acc.astype(jnp.bfloat16)

    r2 = []
    for d in (1, 2, 3):
        p = (me + d) % N_DEV
        rd = pltpu.make_async_remote_copy(
            src_ref=o_ref.at[pl.ds(me * C, C)],
            dst_ref=o_ref.at[pl.ds(me * C, C)],
            send_sem=s2send.at[d],
            recv_sem=s2recv.at[d],
            device_id=(p,),
            device_id_type=pl.DeviceIdType.MESH,
        )
        rd.start()
        r2.append(rd)
    for rd in r2:
        rd.wait()


def _allreduce(partial):
    return pl.pallas_call(
        _ar_body,
        out_shape=jax.ShapeDtypeStruct((SQ, DM), jnp.bfloat16),
        in_specs=[pl.BlockSpec(memory_space=pltpu.VMEM)],
        out_specs=pl.BlockSpec(memory_space=pltpu.VMEM),
        scratch_shapes=[
            pltpu.VMEM((SQ, DM), jnp.bfloat16),
            pltpu.SemaphoreType.DMA((N_DEV,)),
            pltpu.SemaphoreType.DMA((N_DEV,)),
            pltpu.SemaphoreType.DMA((N_DEV,)),
            pltpu.SemaphoreType.DMA((N_DEV,)),
        ],
    )(partial)


def kernel(x, Wq, K_ext, V_ext, Wo):
    xb = x[0].astype(jnp.bfloat16)
    q = xb @ Wq.astype(jnp.bfloat16)
    qh = q.reshape(SQ, HSH, DH).transpose(1, 0, 2)

    kt = K_ext[0].astype(jnp.bfloat16).transpose(1, 0, 2)
    vt = V_ext[0].astype(jnp.bfloat16).transpose(1, 0, 2)

    kf, vf = _a2a(kt, vt)
    ctx = _attn(qh, kf, vf)

    ctx2 = ctx.transpose(1, 0, 2).reshape(SQ, HSH * DH)
    partial = lax.dot_general(ctx2, Wo.astype(jnp.bfloat16),
                              (((1,), (0,)), ((), ())),
                              preferred_element_type=jnp.bfloat16)
    out = _allreduce(partial)
    return out.astype(jnp.float32).reshape(1, SQ, DM)


# device time: 290943 ns/iter; 1.4425x vs baseline; 1.1902x over previous
import jax
import jax.numpy as jnp
from jax import lax
from jax.experimental import pallas as pl
from jax.experimental.pallas import tpu as pltpu

N_DEV = 4
SQ = 2048
SKV_SHARD = 2048
SKV = N_DEV * SKV_SHARD
HSH = 8
DH = 128
DM = 1024
BAND = 128
NGLOB = 32
SCALE = 0.08838834764831843
BANDK = SKV_SHARD + 256


def _a2a_body(kt_ref, vt_ref, kf_ref, vf_ref, ksend, krecv, vsend, vrecv, csem):
    me = lax.axis_index("i")

    lk = pltpu.make_async_copy(
        kt_ref.at[pl.ds(me * HSH, HSH)], kf_ref.at[pl.ds(me * HSH, HSH)],
        csem.at[0])
    lv = pltpu.make_async_copy(
        vt_ref.at[pl.ds(me * HSH, HSH)], vf_ref.at[pl.ds(me * HSH, HSH)],
        csem.at[1])
    lk.start()
    lv.start()

    rdmas = []
    for d in (1, 2, 3):
        p = (me + d) % N_DEV
        rk = pltpu.make_async_remote_copy(
            src_ref=kt_ref.at[pl.ds(p * HSH, HSH)],
            dst_ref=kf_ref.at[pl.ds(me * HSH, HSH)],
            send_sem=ksend.at[d],
            recv_sem=krecv.at[d],
            device_id=(p,),
            device_id_type=pl.DeviceIdType.MESH,
        )
        rv = pltpu.make_async_remote_copy(
            src_ref=vt_ref.at[pl.ds(p * HSH, HSH)],
            dst_ref=vf_ref.at[pl.ds(me * HSH, HSH)],
            send_sem=vsend.at[d],
            recv_sem=vrecv.at[d],
            device_id=(p,),
            device_id_type=pl.DeviceIdType.MESH,
        )
        rk.start()
        rv.start()
        rdmas.append((rk, rv))

    for rk, rv in rdmas:
        rk.wait()
        rv.wait()
    lk.wait()
    lv.wait()


def _a2a(kt, vt):
    return pl.pallas_call(
        _a2a_body,
        out_shape=[
            jax.ShapeDtypeStruct((N_DEV * HSH, SKV_SHARD, DH), jnp.float8_e4m3fn),
            jax.ShapeDtypeStruct((N_DEV * HSH, SKV_SHARD, DH), jnp.bfloat16),
        ],
        in_specs=[
            pl.BlockSpec(memory_space=pl.ANY),
            pl.BlockSpec(memory_space=pl.ANY),
        ],
        out_specs=[
            pl.BlockSpec(memory_space=pl.ANY),
            pl.BlockSpec(memory_space=pl.ANY),
        ],
        scratch_shapes=[
            pltpu.SemaphoreType.DMA((N_DEV,)),
            pltpu.SemaphoreType.DMA((N_DEV,)),
            pltpu.SemaphoreType.DMA((N_DEV,)),
            pltpu.SemaphoreType.DMA((N_DEV,)),
            pltpu.SemaphoreType.DMA((2,)),
        ],
    )(kt, vt)


def _attn_body(q_ref, kf_ref, vf_ref, o_ref):
    kb = jnp.concatenate(
        [kf_ref[0, 0], kf_ref[1, 0, :BANDK - SKV_SHARD]], axis=0
    ).astype(jnp.bfloat16)
    vb = jnp.concatenate(
        [vf_ref[0, 0], vf_ref[1, 0, :BANDK - SKV_SHARD]], axis=0)
    def _scores(q, k, qi0, ki0, glob_only=False):
        s = lax.dot_general(q, k, (((1,), (1,)), ((), ())),
                            preferred_element_type=jnp.float32) * SCALE
        qi = qi0 + lax.broadcasted_iota(jnp.int32, s.shape, 0)
        ki = ki0 + lax.broadcasted_iota(jnp.int32, s.shape, 1)
        if glob_only:
            mask = ki < NGLOB
        else:
            mask = ((jnp.abs(qi - ki) <= BAND) | (ki < NGLOB)) & (qi >= NGLOB)
        return jnp.where(mask, s, -1e9)

    q = q_ref[0, :512, :]
    s = _scores(q, kb[:768], 0, 0)
    m = jnp.max(s, axis=1, keepdims=True)
    w = jnp.exp(s - m)
    w = w / jnp.sum(w, axis=1, keepdims=True)
    ctx = lax.dot_general(w.astype(jnp.bfloat16), vb[:768],
                          (((1,), (0,)), ((), ())),
                          preferred_element_type=jnp.float32)
    o_ref[0, :512, :] = ctx.astype(jnp.bfloat16)

    for qb in (1, 2, 3):
        lo = qb * 512 - BAND
        q = q_ref[0, qb * 512:(qb + 1) * 512, :]
        sg = _scores(q, kb[:128], qb * 512, 0, glob_only=True)
        sb = _scores(q, kb[lo:lo + 768], qb * 512, lo)
        m = jnp.maximum(jnp.max(sg, axis=1, keepdims=True),
                        jnp.max(sb, axis=1, keepdims=True))
        wg = jnp.exp(sg - m)
        wb = jnp.exp(sb - m)
        l = jnp.sum(wg, axis=1, keepdims=True) + jnp.sum(wb, axis=1, keepdims=True)
        ctx = (lax.dot_general(wg.astype(jnp.bfloat16), vb[:128],
                               (((1,), (0,)), ((), ())),
                               preferred_element_type=jnp.float32)
               + lax.dot_general(wb.astype(jnp.bfloat16), vb[lo:lo + 768],
                                 (((1,), (0,)), ((), ())),
                                 preferred_element_type=jnp.float32)) / l
        o_ref[0, qb * 512:(qb + 1) * 512, :] = ctx.astype(jnp.bfloat16)

    kall = jnp.concatenate([kf_ref[s, 0] for s in range(N_DEV)], axis=0).astype(jnp.bfloat16)
    vall = jnp.concatenate([vf_ref[s, 0] for s in range(N_DEV)], axis=0)
    q32 = q_ref[0, :NGLOB, :]
    s2 = lax.dot_general(q32, kall, (((1,), (1,)), ((), ())),
                         preferred_element_type=jnp.float32) * SCALE
    m2 = jnp.max(s2, axis=1, keepdims=True)
    w2 = jnp.exp(s2 - m2)
    w2 = w2 / jnp.sum(w2, axis=1, keepdims=True)
    ctx2 = lax.dot_general(w2.astype(jnp.bfloat16), vall,
                           (((1,), (0,)), ((), ())),
                           preferred_element_type=jnp.float32)
    o_ref[0, :NGLOB, :] = ctx2.astype(jnp.bfloat16)


def _attn(qh, kf, vf):
    kf4 = kf.reshape(N_DEV, HSH, SKV_SHARD, DH)
    vf4 = vf.reshape(N_DEV, HSH, SKV_SHARD, DH)
    return pl.pallas_call(
        _attn_body,
        grid=(HSH,),
        out_shape=jax.ShapeDtypeStruct((HSH, SQ, DH), jnp.bfloat16),
        in_specs=[
            pl.BlockSpec((1, SQ, DH), lambda h: (h, 0, 0)),
            pl.BlockSpec((N_DEV, 1, SKV_SHARD, DH), lambda h: (0, h, 0, 0)),
            pl.BlockSpec((N_DEV, 1, SKV_SHARD, DH), lambda h: (0, h, 0, 0)),
        ],
        out_specs=pl.BlockSpec((1, SQ, DH), lambda h: (h, 0, 0)),
    )(qh, kf4, vf4)


def _ar_body(p_ref, o_ref, rs_ref, s1send, s1recv, s2send, s2recv):
    me = lax.axis_index("i")
    C = SQ // N_DEV

    r1 = []
    for d in (1, 2, 3):
        p = (me + d) % N_DEV
        rd = pltpu.make_async_remote_copy(
            src_ref=p_ref.at[pl.ds(p * C, C)],
            dst_ref=rs_ref.at[pl.ds(d * C, C)],
            send_sem=s1send.at[d],
            recv_sem=s1recv.at[d],
            device_id=(p,),
            device_id_type=pl.DeviceIdType.MESH,
        )
        rd.start()
        r1.append(rd)

    acc = p_ref[pl.ds(me * C, C), :].astype(jnp.float32)
    for d, rd in zip((1, 2, 3), r1):
        rd.wait()
        acc = acc + rs_ref[d * C:(d + 1) * C, :].astype(jnp.float32)
    o_ref[pl.ds(me * C, C), :] = acc.astype(jnp.bfloat16)

    r2 = []
    for d in (1, 2, 3):
        p = (me + d) % N_DEV
        rd = pltpu.make_async_remote_copy(
            src_ref=o_ref.at[pl.ds(me * C, C)],
            dst_ref=o_ref.at[pl.ds(me * C, C)],
            send_sem=s2send.at[d],
            recv_sem=s2recv.at[d],
            device_id=(p,),
            device_id_type=pl.DeviceIdType.MESH,
        )
        rd.start()
        r2.append(rd)
    for rd in r2:
        rd.wait()


def _allreduce(partial):
    return pl.pallas_call(
        _ar_body,
        out_shape=jax.ShapeDtypeStruct((SQ, DM), jnp.bfloat16),
        in_specs=[pl.BlockSpec(memory_space=pltpu.VMEM)],
        out_specs=pl.BlockSpec(memory_space=pltpu.VMEM),
        scratch_shapes=[
            pltpu.VMEM((SQ, DM), jnp.bfloat16),
            pltpu.SemaphoreType.DMA((N_DEV,)),
            pltpu.SemaphoreType.DMA((N_DEV,)),
            pltpu.SemaphoreType.DMA((N_DEV,)),
            pltpu.SemaphoreType.DMA((N_DEV,)),
        ],
    )(partial)


def kernel(x, Wq, K_ext, V_ext, Wo):
    xb = x[0].astype(jnp.bfloat16)
    q = xb @ Wq.astype(jnp.bfloat16)
    qh = q.reshape(SQ, HSH, DH).transpose(1, 0, 2)

    kt = K_ext[0].astype(jnp.float8_e4m3fn).transpose(1, 0, 2)
    vt = V_ext[0].astype(jnp.bfloat16).transpose(1, 0, 2)

    kf, vf = _a2a(kt, vt)
    ctx = _attn(qh, kf, vf)

    ctx2 = ctx.transpose(1, 0, 2).reshape(SQ, HSH * DH)
    partial = lax.dot_general(ctx2, Wo.astype(jnp.bfloat16),
                              (((1,), (0,)), ((), ())),
                              preferred_element_type=jnp.bfloat16)
    out = _allreduce(partial)
    return out.astype(jnp.float32).reshape(1, SQ, DM)


# device time: 288564 ns/iter; 1.4544x vs baseline; 1.0082x over previous
import jax
import jax.numpy as jnp
from jax import lax
from jax.experimental import pallas as pl
from jax.experimental.pallas import tpu as pltpu

N_DEV = 4
SQ = 2048
SKV_SHARD = 2048
HSH = 8
NH = 32
DH = 128
DM = 1024
BAND = 128
NGLOB = 32
SCALE = 0.08838834764831843
SLIV = 128
BK = SKV_SHARD + SLIV


def _exch_body(kt_ref, vt_ref, q32_ref,
               kb_ref, vb_ref, g32_ref,
               q32all, kc, vc, osend, stsend, oall, stall, rlk, rlv,
               qsend, qrecv, opsend, oprecv, stpsend, stprecv,
               bsend, brecv, rsend, rrecv, fsend, frecv,
               svsend, svrecv, lsem, cksem, cvsem):
    me = lax.axis_index("i")

    lq = pltpu.make_async_copy(q32_ref, q32all.at[pl.ds(me * HSH, HSH)],
                               lsem.at[2])
    lq.start()
    qr = []
    for d in (1, 2, 3):
        p = (me + d) % N_DEV
        r = pltpu.make_async_remote_copy(
            src_ref=q32_ref,
            dst_ref=q32all.at[pl.ds(me * HSH, HSH)],
            send_sem=qsend.at[d], recv_sem=qrecv.at[d],
            device_id=(p,), device_id_type=pl.DeviceIdType.MESH)
        r.start()
        qr.append(r)

    band0 = [
        pltpu.make_async_remote_copy(
            src_ref=kt_ref.at[pl.ds(16, 4)], dst_ref=rlk,
            send_sem=rsend.at[0], recv_sem=rrecv.at[0],
            device_id=(1,), device_id_type=pl.DeviceIdType.MESH),
        pltpu.make_async_remote_copy(
            src_ref=vt_ref.at[pl.ds(16, 4)], dst_ref=rlv,
            send_sem=rsend.at[1], recv_sem=rrecv.at[1],
            device_id=(1,), device_id_type=pl.DeviceIdType.MESH),
        pltpu.make_async_remote_copy(
            src_ref=kt_ref.at[pl.ds(20, 4)], dst_ref=rlk,
            send_sem=rsend.at[2], recv_sem=rrecv.at[0],
            device_id=(3,), device_id_type=pl.DeviceIdType.MESH),
        pltpu.make_async_remote_copy(
            src_ref=vt_ref.at[pl.ds(20, 4)], dst_ref=rlv,
            send_sem=rsend.at[3], recv_sem=rrecv.at[1],
            device_id=(3,), device_id_type=pl.DeviceIdType.MESH),
        pltpu.make_async_remote_copy(
            src_ref=kt_ref.at[pl.ds(8, 8)],
            dst_ref=kb_ref.at[:, pl.ds(0, SKV_SHARD)],
            send_sem=bsend.at[0], recv_sem=brecv.at[0],
            device_id=(1,), device_id_type=pl.DeviceIdType.MESH),
        pltpu.make_async_remote_copy(
            src_ref=vt_ref.at[pl.ds(8, 8)],
            dst_ref=vb_ref.at[:, pl.ds(0, SKV_SHARD)],
            send_sem=bsend.at[1], recv_sem=brecv.at[1],
            device_id=(1,), device_id_type=pl.DeviceIdType.MESH),
        pltpu.make_async_remote_copy(
            src_ref=kt_ref.at[pl.ds(24, 8)],
            dst_ref=kb_ref.at[:, pl.ds(0, SKV_SHARD)],
            send_sem=bsend.at[2], recv_sem=brecv.at[0],
            device_id=(3,), device_id_type=pl.DeviceIdType.MESH),
        pltpu.make_async_remote_copy(
            src_ref=vt_ref.at[pl.ds(24, 8)],
            dst_ref=vb_ref.at[:, pl.ds(0, SKV_SHARD)],
            send_sem=bsend.at[3], recv_sem=brecv.at[1],
            device_id=(3,), device_id_type=pl.DeviceIdType.MESH),
    ]
    loc0 = [
        pltpu.make_async_copy(kt_ref.at[pl.ds(0, HSH)],
                              kb_ref.at[:, pl.ds(0, SKV_SHARD)], lsem.at[0]),
        pltpu.make_async_copy(vt_ref.at[pl.ds(0, HSH)],
                              vb_ref.at[:, pl.ds(0, SKV_SHARD)], lsem.at[1]),
    ]

    sliv1 = []
    for i, p in enumerate((0, 2, 3)):
        sliv1.append(pltpu.make_async_remote_copy(
            src_ref=kt_ref.at[pl.ds(p * HSH, HSH), pl.ds(0, SLIV)],
            dst_ref=kb_ref.at[:, pl.ds(SKV_SHARD, SLIV)],
            send_sem=svsend.at[2 * i], recv_sem=svrecv.at[0],
            device_id=(p,), device_id_type=pl.DeviceIdType.MESH))
        sliv1.append(pltpu.make_async_remote_copy(
            src_ref=vt_ref.at[pl.ds(p * HSH, HSH), pl.ds(0, SLIV)],
            dst_ref=vb_ref.at[:, pl.ds(SKV_SHARD, SLIV)],
            send_sem=svsend.at[2 * i + 1], recv_sem=svrecv.at[1],
            device_id=(p,), device_id_type=pl.DeviceIdType.MESH))
    loc1 = [
        pltpu.make_async_copy(
            kt_ref.at[pl.ds(HSH, HSH), pl.ds(0, SLIV)],
            kb_ref.at[:, pl.ds(SKV_SHARD, SLIV)], lsem.at[0]),
        pltpu.make_async_copy(
            vt_ref.at[pl.ds(HSH, HSH), pl.ds(0, SLIV)],
            vb_ref.at[:, pl.ds(SKV_SHARD, SLIV)], lsem.at[1]),
    ]

    relay_recv = [
        pltpu.make_async_remote_copy(
            src_ref=rlk, dst_ref=rlk, send_sem=lsem.at[3],
            recv_sem=rrecv.at[0], device_id=(0,),
            device_id_type=pl.DeviceIdType.MESH),
        pltpu.make_async_remote_copy(
            src_ref=rlv, dst_ref=rlv, send_sem=lsem.at[3],
            recv_sem=rrecv.at[1], device_id=(0,),
            device_id_type=pl.DeviceIdType.MESH),
    ]
    fwd1 = [
        pltpu.make_async_remote_copy(
            src_ref=rlk, dst_ref=kb_ref.at[pl.ds(0, 4), pl.ds(0, SKV_SHARD)],
            send_sem=fsend.at[0], recv_sem=frecv.at[0],
            device_id=(2,), device_id_type=pl.DeviceIdType.MESH),
        pltpu.make_async_remote_copy(
            src_ref=rlv, dst_ref=vb_ref.at[pl.ds(0, 4), pl.ds(0, SKV_SHARD)],
            send_sem=fsend.at[1], recv_sem=frecv.at[1],
            device_id=(2,), device_id_type=pl.DeviceIdType.MESH),
    ]
    fwd3 = [
        pltpu.make_async_remote_copy(
            src_ref=rlk, dst_ref=kb_ref.at[pl.ds(4, 4), pl.ds(0, SKV_SHARD)],
            send_sem=fsend.at[0], recv_sem=frecv.at[2],
            device_id=(2,), device_id_type=pl.DeviceIdType.MESH),
        pltpu.make_async_remote_copy(
            src_ref=rlv, dst_ref=vb_ref.at[pl.ds(4, 4), pl.ds(0, SKV_SHARD)],
            send_sem=fsend.at[1], recv_sem=frecv.at[3],
            device_id=(2,), device_id_type=pl.DeviceIdType.MESH),
    ]

    @pl.when(me == 0)
    def _():
        for r in band0:
            r.start()
        for c in loc0:
            c.start()

    @pl.when(me == 1)
    def _():
        for r in sliv1:
            r.start()
        for c in loc1:
            c.start()

    @pl.when(me == 1)
    def _():
        relay_recv[0].wait_recv()
        relay_recv[1].wait_recv()
        for r in fwd1:
            r.start()

    @pl.when(me == 3)
    def _():
        relay_recv[0].wait_recv()
        relay_recv[1].wait_recv()
        for r in fwd3:
            r.start()

    for r in qr:
        r.wait()
    lq.wait()
    qv = q32all[...]

    ck0 = pltpu.make_async_copy(kt_ref.at[pl.ds(0, HSH)], kc.at[0], cksem.at[0])
    cv0 = pltpu.make_async_copy(vt_ref.at[pl.ds(0, HSH)], vc.at[0], cvsem.at[0])
    ck0.start()
    cv0.start()
    copies = [(ck0, cv0)]
    for hc in range(N_DEV):
        if hc + 1 < N_DEV:
            nk = pltpu.make_async_copy(
                kt_ref.at[pl.ds((hc + 1) * HSH, HSH)], kc.at[(hc + 1) % 2],
                cksem.at[(hc + 1) % 2])
            nv = pltpu.make_async_copy(
                vt_ref.at[pl.ds((hc + 1) * HSH, HSH)], vc.at[(hc + 1) % 2],
                cvsem.at[(hc + 1) % 2])
            nk.start()
            nv.start()
            copies.append((nk, nv))
        copies[hc][0].wait()
        copies[hc][1].wait()
        qc = qv[hc * HSH:(hc + 1) * HSH]
        s = lax.dot_general(qc, kc[hc % 2], (((2,), (2,)), ((0,), (0,))),
                            preferred_element_type=jnp.float32) * SCALE
        m_c = jnp.max(s, axis=2)
        w = jnp.exp(s - m_c[:, :, None])
        l_c = jnp.sum(w, axis=2)
        o_c = lax.dot_general(w.astype(jnp.bfloat16), vc[hc % 2],
                              (((2,), (1,)), ((0,), (0,))),
                              preferred_element_type=jnp.float32)
        osend[hc] = o_c
        stsend[hc, 0] = m_c
        stsend[hc, 1] = l_c

    oall[pl.ds(me, 1)] = osend[pl.ds(me, 1)]
    stall[pl.ds(me, 1)] = stsend[pl.ds(me, 1)]
    pr = []
    for d in (1, 2, 3):
        p = (me + d) % N_DEV
        r1 = pltpu.make_async_remote_copy(
            src_ref=osend.at[pl.ds(p, 1)], dst_ref=oall.at[pl.ds(me, 1)],
            send_sem=opsend.at[d], recv_sem=oprecv.at[d],
            device_id=(p,), device_id_type=pl.DeviceIdType.MESH)
        r2 = pltpu.make_async_remote_copy(
            src_ref=stsend.at[pl.ds(p, 1)], dst_ref=stall.at[pl.ds(me, 1)],
            send_sem=stpsend.at[d], recv_sem=stprecv.at[d],
            device_id=(p,), device_id_type=pl.DeviceIdType.MESH)
        r1.start()
        r2.start()
        pr.append((r1, r2))
    for r1, r2 in pr:
        r1.wait()
        r2.wait()

    ov = oall[...]
    stv = stall[...]
    mj = stv[:, 0]
    mm = jnp.max(mj, axis=0)
    a = jnp.exp(mj - mm[None])
    ctx = jnp.sum(a[..., None] * ov, axis=0)
    ll = jnp.sum(a * stv[:, 1], axis=0)
    g32_ref[...] = (ctx / ll[..., None]).astype(jnp.bfloat16)

    def _recv(dst, sem):
        return pltpu.make_async_remote_copy(
            src_ref=dst, dst_ref=dst, send_sem=lsem.at[3], recv_sem=sem,
            device_id=(0,), device_id_type=pl.DeviceIdType.MESH)

    @pl.when(me == 0)
    def _():
        _recv(kb_ref.at[:, pl.ds(SKV_SHARD, SLIV)], svrecv.at[0]).wait_recv()
        _recv(vb_ref.at[:, pl.ds(SKV_SHARD, SLIV)], svrecv.at[1]).wait_recv()
        for r in band0:
            r.wait_send()
        for c in loc0:
            c.wait()

    @pl.when(me == 1)
    def _():
        _recv(kb_ref.at[:, pl.ds(0, SKV_SHARD)], brecv.at[0]).wait_recv()
        _recv(vb_ref.at[:, pl.ds(0, SKV_SHARD)], brecv.at[1]).wait_recv()
        for r in fwd1:
            r.wait_send()
        for r in sliv1:
            r.wait_send()
        for c in loc1:
            c.wait()

    @pl.when(me == 3)
    def _():
        _recv(kb_ref.at[:, pl.ds(0, SKV_SHARD)], brecv.at[0]).wait_recv()
        _recv(vb_ref.at[:, pl.ds(0, SKV_SHARD)], brecv.at[1]).wait_recv()
        for r in fwd3:
            r.wait_send()
        _recv(kb_ref.at[:, pl.ds(SKV_SHARD, SLIV)], svrecv.at[0]).wait_recv()
        _recv(vb_ref.at[:, pl.ds(SKV_SHARD, SLIV)], svrecv.at[1]).wait_recv()

    @pl.when(me == 2)
    def _():
        _recv(kb_ref.at[pl.ds(0, 4), pl.ds(0, SKV_SHARD)], frecv.at[0]).wait_recv()
        _recv(vb_ref.at[pl.ds(0, 4), pl.ds(0, SKV_SHARD)], frecv.at[1]).wait_recv()
        _recv(kb_ref.at[pl.ds(4, 4), pl.ds(0, SKV_SHARD)], frecv.at[2]).wait_recv()
        _recv(vb_ref.at[pl.ds(4, 4), pl.ds(0, SKV_SHARD)], frecv.at[3]).wait_recv()
        _recv(kb_ref.at[:, pl.ds(SKV_SHARD, SLIV)], svrecv.at[0]).wait_recv()
        _recv(vb_ref.at[:, pl.ds(SKV_SHARD, SLIV)], svrecv.at[1]).wait_recv()


def _exchange(kt, vt, q32):
    return pl.pallas_call(
        _exch_body,
        out_shape=[
            jax.ShapeDtypeStruct((HSH, BK, DH), jnp.bfloat16),
            jax.ShapeDtypeStruct((HSH, BK, DH), jnp.bfloat16),
            jax.ShapeDtypeStruct((HSH, NGLOB, DH), jnp.bfloat16),
        ],
        in_specs=[
            pl.BlockSpec(memory_space=pl.ANY),
            pl.BlockSpec(memory_space=pl.ANY),
            pl.BlockSpec(memory_space=pltpu.VMEM),
        ],
        out_specs=[
            pl.BlockSpec(memory_space=pl.ANY),
            pl.BlockSpec(memory_space=pl.ANY),
            pl.BlockSpec(memory_space=pltpu.VMEM),
        ],
        scratch_shapes=[
            pltpu.VMEM((NH, NGLOB, DH), jnp.bfloat16),
            pltpu.VMEM((2, HSH, SKV_SHARD, DH), jnp.bfloat16),
            pltpu.VMEM((2, HSH, SKV_SHARD, DH), jnp.bfloat16),
            pltpu.VMEM((N_DEV, HSH, NGLOB, DH), jnp.float32),
            pltpu.VMEM((N_DEV, 2, HSH, NGLOB), jnp.float32),
            pltpu.VMEM((N_DEV, HSH, NGLOB, DH), jnp.float32),
            pltpu.VMEM((N_DEV, 2, HSH, NGLOB), jnp.float32),
            pltpu.VMEM((4, SKV_SHARD, DH), jnp.bfloat16),
            pltpu.VMEM((4, SKV_SHARD, DH), jnp.bfloat16),
            pltpu.SemaphoreType.DMA((N_DEV,)),
            pltpu.SemaphoreType.DMA((N_DEV,)),
            pltpu.SemaphoreType.DMA((N_DEV,)),
            pltpu.SemaphoreType.DMA((N_DEV,)),
            pltpu.SemaphoreType.DMA((N_DEV,)),
            pltpu.SemaphoreType.DMA((N_DEV,)),
            pltpu.SemaphoreType.DMA((4,)),
            pltpu.SemaphoreType.DMA((2,)),
            pltpu.SemaphoreType.DMA((4,)),
            pltpu.SemaphoreType.DMA((2,)),
            pltpu.SemaphoreType.DMA((2,)),
            pltpu.SemaphoreType.DMA((4,)),
            pltpu.SemaphoreType.DMA((8,)),
            pltpu.SemaphoreType.DMA((2,)),
            pltpu.SemaphoreType.DMA((4,)),
            pltpu.SemaphoreType.DMA((2,)),
            pltpu.SemaphoreType.DMA((2,)),
        ],
    )(kt, vt, q32)


def _attn_body(q_ref, kb_ref, vb_ref, g32_ref, o_ref):
    kb = kb_ref[0]
    vb = vb_ref[0]

    def _scores(q, k, qi0, ki0, glob_only=False):
        s = lax.dot_general(q, k, (((1,), (1,)), ((), ())),
                            preferred_element_type=jnp.float32) * SCALE
        qi = qi0 + lax.broadcasted_iota(jnp.int32, s.shape, 0)
        ki = ki0 + lax.broadcasted_iota(jnp.int32, s.shape, 1)
        if glob_only:
            mask = ki < NGLOB
        else:
            mask = ((jnp.abs(qi - ki) <= BAND) | (ki < NGLOB)) & (qi >= NGLOB)
        return jnp.where(mask, s, -1e9)

    q = q_ref[0, :512, :]
    s = _scores(q, kb[:768], 0, 0)
    m = jnp.max(s, axis=1, keepdims=True)
    w = jnp.exp(s - m)
    w = w / jnp.sum(w, axis=1, keepdims=True)
    ctx = lax.dot_general(w.astype(jnp.bfloat16), vb[:768],
                          (((1,), (0,)), ((), ())),
                          preferred_element_type=jnp.float32)
    o_ref[0, :512, :] = ctx.astype(jnp.bfloat16)

    for qb in (1, 2, 3):
        lo = qb * 512 - BAND
        q = q_ref[0, qb * 512:(qb + 1) * 512, :]
        sg = _scores(q, kb[:128], qb * 512, 0, glob_only=True)
        sb = _scores(q, kb[lo:lo + 768], qb * 512, lo)
        m = jnp.maximum(jnp.max(sg, axis=1, keepdims=True),
                        jnp.max(sb, axis=1, keepdims=True))
        wg = jnp.exp(sg - m)
        wb = jnp.exp(sb - m)
        l = jnp.sum(wg, axis=1, keepdims=True) + jnp.sum(wb, axis=1, keepdims=True)
        ctx = (lax.dot_general(wg.astype(jnp.bfloat16), vb[:128],
                               (((1,), (0,)), ((), ())),
                               preferred_element_type=jnp.float32)
               + lax.dot_general(wb.astype(jnp.bfloat16), vb[lo:lo + 768],
                                 (((1,), (0,)), ((), ())),
                                 preferred_element_type=jnp.float32)) / l
        o_ref[0, qb * 512:(qb + 1) * 512, :] = ctx.astype(jnp.bfloat16)

    o_ref[0, :NGLOB, :] = g32_ref[0]


def _attn(qh, kb, vb, g32):
    return pl.pallas_call(
        _attn_body,
        grid=(HSH,),
        out_shape=jax.ShapeDtypeStruct((HSH, SQ, DH), jnp.bfloat16),
        in_specs=[
            pl.BlockSpec((1, SQ, DH), lambda h: (h, 0, 0)),
            pl.BlockSpec((1, BK, DH), lambda h: (h, 0, 0)),
            pl.BlockSpec((1, BK, DH), lambda h: (h, 0, 0)),
            pl.BlockSpec((1, NGLOB, DH), lambda h: (h, 0, 0)),
        ],
        out_specs=pl.BlockSpec((1, SQ, DH), lambda h: (h, 0, 0)),
    )(qh, kb, vb, g32)


def _ar_body(p_ref, o_ref, rs_ref, s1send, s1recv, s2send, s2recv):
    me = lax.axis_index("i")
    C = SQ // N_DEV

    r1 = []
    for d in (1, 2, 3):
        p = (me + d) % N_DEV
        rd = pltpu.make_async_remote_copy(
            src_ref=p_ref.at[pl.ds(p * C, C)],
            dst_ref=rs_ref.at[pl.ds(d * C, C)],
            send_sem=s1send.at[d],
            recv_sem=s1recv.at[d],
            device_id=(p,),
            device_id_type=pl.DeviceIdType.MESH,
        )
        rd.start()
        r1.append(rd)

    acc = p_ref[pl.ds(me * C, C), :].astype(jnp.float32)
    for d, rd in zip((1, 2, 3), r1):
        rd.wait()
        acc = acc + rs_ref[d * C:(d + 1) * C, :].astype(jnp.float32)
    o_ref[pl.ds(me * C, C), :] = acc.astype(jnp.bfloat16)

    r2 = []
    for d in (1, 2, 3):
        p = (me + d) % N_DEV
        rd = pltpu.make_async_remote_copy(
            src_ref=o_ref.at[pl.ds(me * C, C)],
            dst_ref=o_ref.at[pl.ds(me * C, C)],
            send_sem=s2send.at[d],
            recv_sem=s2recv.at[d],
            device_id=(p,),
            device_id_type=pl.DeviceIdType.MESH,
        )
        rd.start()
        r2.append(rd)
    for rd in r2:
        rd.wait()


def _allreduce(partial):
    return pl.pallas_call(
        _ar_body,
        out_shape=jax.ShapeDtypeStruct((SQ, DM), jnp.bfloat16),
        in_specs=[pl.BlockSpec(memory_space=pltpu.VMEM)],
        out_specs=pl.BlockSpec(memory_space=pltpu.VMEM),
        scratch_shapes=[
            pltpu.VMEM((SQ, DM), jnp.bfloat16),
            pltpu.SemaphoreType.DMA((N_DEV,)),
            pltpu.SemaphoreType.DMA((N_DEV,)),
            pltpu.SemaphoreType.DMA((N_DEV,)),
            pltpu.SemaphoreType.DMA((N_DEV,)),
        ],
    )(partial)


def kernel(x, Wq, K_ext, V_ext, Wo):
    xb = x[0].astype(jnp.bfloat16)
    q = xb @ Wq.astype(jnp.bfloat16)
    qh = q.reshape(SQ, HSH, DH).transpose(1, 0, 2)
    q32 = qh[:, :NGLOB, :]

    kt = K_ext[0].astype(jnp.bfloat16).transpose(1, 0, 2)
    vt = V_ext[0].astype(jnp.bfloat16).transpose(1, 0, 2)

    kb, vb, g32 = _exchange(kt, vt, q32)
    ctx = _attn(qh, kb, vb, g32)

    ctx2 = ctx.transpose(1, 0, 2).reshape(SQ, HSH * DH)
    partial = lax.dot_general(ctx2, Wo.astype(jnp.bfloat16),
                              (((1,), (0,)), ((), ())),
                              preferred_element_type=jnp.bfloat16)
    out = _allreduce(partial)
    return out.astype(jnp.float32).reshape(1, SQ, DM)


# device time: 286335 ns/iter; 1.4658x vs baseline; 1.0078x over previous
import jax
import jax.numpy as jnp
from jax import lax
from jax.experimental import pallas as pl
from jax.experimental.pallas import tpu as pltpu

N_DEV = 4
SQ = 2048
SKV_SHARD = 2048
HSH = 8
NH = 32
DH = 128
DM = 1024
BAND = 128
NGLOB = 32
SCALE = 0.08838834764831843
SLIV = 128
BK = SKV_SHARD + SLIV


def _exch_body(kt_ref, vt_ref, q32_ref,
               kb_ref, vb_ref, g32_ref,
               q32all, kc, vc, osend, stsend, oall, stall, rlk, rlv,
               qsend, qrecv, opsend, oprecv, stpsend, stprecv,
               bsend, brecv, rsend, rrecv, fsend, frecv,
               svsend, svrecv, lsem, cksem, cvsem):
    me = lax.axis_index("i")

    lq = pltpu.make_async_copy(q32_ref, q32all.at[pl.ds(me * HSH, HSH)],
                               lsem.at[2])
    lq.start()
    qr = []
    for d in (1, 2, 3):
        p = (me + d) % N_DEV
        r = pltpu.make_async_remote_copy(
            src_ref=q32_ref,
            dst_ref=q32all.at[pl.ds(me * HSH, HSH)],
            send_sem=qsend.at[d], recv_sem=qrecv.at[d],
            device_id=(p,), device_id_type=pl.DeviceIdType.MESH)
        r.start()
        qr.append(r)

    band0 = [
        pltpu.make_async_remote_copy(
            src_ref=kt_ref.at[pl.ds(16, 4)], dst_ref=rlk,
            send_sem=rsend.at[0], recv_sem=rrecv.at[0],
            device_id=(1,), device_id_type=pl.DeviceIdType.MESH),
        pltpu.make_async_remote_copy(
            src_ref=vt_ref.at[pl.ds(16, 4)], dst_ref=rlv,
            send_sem=rsend.at[1], recv_sem=rrecv.at[1],
            device_id=(1,), device_id_type=pl.DeviceIdType.MESH),
        pltpu.make_async_remote_copy(
            src_ref=kt_ref.at[pl.ds(20, 4)], dst_ref=rlk,
            send_sem=rsend.at[2], recv_sem=rrecv.at[0],
            device_id=(3,), device_id_type=pl.DeviceIdType.MESH),
        pltpu.make_async_remote_copy(
            src_ref=vt_ref.at[pl.ds(20, 4)], dst_ref=rlv,
            send_sem=rsend.at[3], recv_sem=rrecv.at[1],
            device_id=(3,), device_id_type=pl.DeviceIdType.MESH),
        pltpu.make_async_remote_copy(
            src_ref=kt_ref.at[pl.ds(8, 8)],
            dst_ref=kb_ref.at[:, pl.ds(0, SKV_SHARD)],
            send_sem=bsend.at[0], recv_sem=brecv.at[0],
            device_id=(1,), device_id_type=pl.DeviceIdType.MESH),
        pltpu.make_async_remote_copy(
            src_ref=vt_ref.at[pl.ds(8, 8)],
            dst_ref=vb_ref.at[:, pl.ds(0, SKV_SHARD)],
            send_sem=bsend.at[1], recv_sem=brecv.at[1],
            device_id=(1,), device_id_type=pl.DeviceIdType.MESH),
        pltpu.make_async_remote_copy(
            src_ref=kt_ref.at[pl.ds(24, 8)],
            dst_ref=kb_ref.at[:, pl.ds(0, SKV_SHARD)],
            send_sem=bsend.at[2], recv_sem=brecv.at[0],
            device_id=(3,), device_id_type=pl.DeviceIdType.MESH),
        pltpu.make_async_remote_copy(
            src_ref=vt_ref.at[pl.ds(24, 8)],
            dst_ref=vb_ref.at[:, pl.ds(0, SKV_SHARD)],
            send_sem=bsend.at[3], recv_sem=brecv.at[1],
            device_id=(3,), device_id_type=pl.DeviceIdType.MESH),
    ]
    loc0 = [
        pltpu.make_async_copy(kt_ref.at[pl.ds(0, HSH)],
                              kb_ref.at[:, pl.ds(0, SKV_SHARD)], lsem.at[0]),
        pltpu.make_async_copy(vt_ref.at[pl.ds(0, HSH)],
                              vb_ref.at[:, pl.ds(0, SKV_SHARD)], lsem.at[1]),
    ]

    sliv1 = []
    for i, p in enumerate((0, 2, 3)):
        sliv1.append(pltpu.make_async_remote_copy(
            src_ref=kt_ref.at[pl.ds(p * HSH, HSH), pl.ds(0, SLIV)],
            dst_ref=kb_ref.at[:, pl.ds(SKV_SHARD, SLIV)],
            send_sem=svsend.at[2 * i], recv_sem=svrecv.at[0],
            device_id=(p,), device_id_type=pl.DeviceIdType.MESH))
        sliv1.append(pltpu.make_async_remote_copy(
            src_ref=vt_ref.at[pl.ds(p * HSH, HSH), pl.ds(0, SLIV)],
            dst_ref=vb_ref.at[:, pl.ds(SKV_SHARD, SLIV)],
            send_sem=svsend.at[2 * i + 1], recv_sem=svrecv.at[1],
            device_id=(p,), device_id_type=pl.DeviceIdType.MESH))
    loc1 = [
        pltpu.make_async_copy(
            kt_ref.at[pl.ds(HSH, HSH), pl.ds(0, SLIV)],
            kb_ref.at[:, pl.ds(SKV_SHARD, SLIV)], lsem.at[0]),
        pltpu.make_async_copy(
            vt_ref.at[pl.ds(HSH, HSH), pl.ds(0, SLIV)],
            vb_ref.at[:, pl.ds(SKV_SHARD, SLIV)], lsem.at[1]),
    ]

    relay_recv = [
        pltpu.make_async_remote_copy(
            src_ref=rlk, dst_ref=rlk, send_sem=lsem.at[3],
            recv_sem=rrecv.at[0], device_id=(0,),
            device_id_type=pl.DeviceIdType.MESH),
        pltpu.make_async_remote_copy(
            src_ref=rlv, dst_ref=rlv, send_sem=lsem.at[3],
            recv_sem=rrecv.at[1], device_id=(0,),
            device_id_type=pl.DeviceIdType.MESH),
    ]
    fwd1 = [
        pltpu.make_async_remote_copy(
            src_ref=rlk, dst_ref=kb_ref.at[pl.ds(0, 4), pl.ds(0, SKV_SHARD)],
            send_sem=fsend.at[0], recv_sem=frecv.at[0],
            device_id=(2,), device_id_type=pl.DeviceIdType.MESH),
        pltpu.make_async_remote_copy(
            src_ref=rlv, dst_ref=vb_ref.at[pl.ds(0, 4), pl.ds(0, SKV_SHARD)],
            send_sem=fsend.at[1], recv_sem=frecv.at[1],
            device_id=(2,), device_id_type=pl.DeviceIdType.MESH),
    ]
    fwd3 = [
        pltpu.make_async_remote_copy(
            src_ref=rlk, dst_ref=kb_ref.at[pl.ds(4, 4), pl.ds(0, SKV_SHARD)],
            send_sem=fsend.at[0], recv_sem=frecv.at[2],
            device_id=(2,), device_id_type=pl.DeviceIdType.MESH),
        pltpu.make_async_remote_copy(
            src_ref=rlv, dst_ref=vb_ref.at[pl.ds(4, 4), pl.ds(0, SKV_SHARD)],
            send_sem=fsend.at[1], recv_sem=frecv.at[3],
            device_id=(2,), device_id_type=pl.DeviceIdType.MESH),
    ]

    @pl.when(me == 0)
    def _():
        for r in band0:
            r.start()
        for c in loc0:
            c.start()

    @pl.when(me == 1)
    def _():
        for r in sliv1:
            r.start()
        for c in loc1:
            c.start()

    @pl.when(me == 1)
    def _():
        relay_recv[0].wait_recv()
        relay_recv[1].wait_recv()
        for r in fwd1:
            r.start()

    @pl.when(me == 3)
    def _():
        relay_recv[0].wait_recv()
        relay_recv[1].wait_recv()
        for r in fwd3:
            r.start()

    for r in qr:
        r.wait()
    lq.wait()
    qv = q32all[...]

    ck0 = pltpu.make_async_copy(kt_ref.at[pl.ds(0, HSH)], kc.at[0], cksem.at[0])
    cv0 = pltpu.make_async_copy(vt_ref.at[pl.ds(0, HSH)], vc.at[0], cvsem.at[0])
    ck0.start()
    cv0.start()
    copies = [(ck0, cv0)]
    for hc in range(N_DEV):
        if hc + 1 < N_DEV:
            nk = pltpu.make_async_copy(
                kt_ref.at[pl.ds((hc + 1) * HSH, HSH)], kc.at[(hc + 1) % 2],
                cksem.at[(hc + 1) % 2])
            nv = pltpu.make_async_copy(
                vt_ref.at[pl.ds((hc + 1) * HSH, HSH)], vc.at[(hc + 1) % 2],
                cvsem.at[(hc + 1) % 2])
            nk.start()
            nv.start()
            copies.append((nk, nv))
        copies[hc][0].wait()
        copies[hc][1].wait()
        qc = qv[hc * HSH:(hc + 1) * HSH]
        s = lax.dot_general(qc, kc[hc % 2], (((2,), (2,)), ((0,), (0,))),
                            preferred_element_type=jnp.float32) * SCALE
        m_c = jnp.max(s, axis=2)
        w = jnp.exp(s - m_c[:, :, None])
        l_c = jnp.sum(w, axis=2)
        o_c = lax.dot_general(w.astype(jnp.bfloat16), vc[hc % 2],
                              (((2,), (1,)), ((0,), (0,))),
                              preferred_element_type=jnp.float32)
        osend[hc] = o_c
        stsend[hc, 0] = m_c
        stsend[hc, 1] = l_c

    oall[pl.ds(me, 1)] = osend[pl.ds(me, 1)]
    stall[pl.ds(me, 1)] = stsend[pl.ds(me, 1)]
    pr = []
    for d in (1, 2, 3):
        p = (me + d) % N_DEV
        r1 = pltpu.make_async_remote_copy(
            src_ref=osend.at[pl.ds(p, 1)], dst_ref=oall.at[pl.ds(me, 1)],
            send_sem=opsend.at[d], recv_sem=oprecv.at[d],
            device_id=(p,), device_id_type=pl.DeviceIdType.MESH)
        r2 = pltpu.make_async_remote_copy(
            src_ref=stsend.at[pl.ds(p, 1)], dst_ref=stall.at[pl.ds(me, 1)],
            send_sem=stpsend.at[d], recv_sem=stprecv.at[d],
            device_id=(p,), device_id_type=pl.DeviceIdType.MESH)
        r1.start()
        r2.start()
        pr.append((r1, r2))
    for r1, r2 in pr:
        r1.wait()
        r2.wait()

    ov = oall[...]
    stv = stall[...]
    mj = stv[:, 0]
    mm = jnp.max(mj, axis=0)
    a = jnp.exp(mj - mm[None])
    ctx = jnp.sum(a[..., None] * ov, axis=0)
    ll = jnp.sum(a * stv[:, 1], axis=0)
    g32_ref[...] = (ctx / ll[..., None]).astype(jnp.bfloat16)

    def _recv(dst, sem):
        return pltpu.make_async_remote_copy(
            src_ref=dst, dst_ref=dst, send_sem=lsem.at[3], recv_sem=sem,
            device_id=(0,), device_id_type=pl.DeviceIdType.MESH)

    @pl.when(me == 0)
    def _():
        _recv(kb_ref.at[:, pl.ds(SKV_SHARD, SLIV)], svrecv.at[0]).wait_recv()
        _recv(vb_ref.at[:, pl.ds(SKV_SHARD, SLIV)], svrecv.at[1]).wait_recv()
        for r in band0:
            r.wait_send()
        for c in loc0:
            c.wait()

    @pl.when(me == 1)
    def _():
        _recv(kb_ref.at[:, pl.ds(0, SKV_SHARD)], brecv.at[0]).wait_recv()
        _recv(vb_ref.at[:, pl.ds(0, SKV_SHARD)], brecv.at[1]).wait_recv()
        for r in fwd1:
            r.wait_send()
        for r in sliv1:
            r.wait_send()
        for c in loc1:
            c.wait()

    @pl.when(me == 3)
    def _():
        _recv(kb_ref.at[:, pl.ds(0, SKV_SHARD)], brecv.at[0]).wait_recv()
        _recv(vb_ref.at[:, pl.ds(0, SKV_SHARD)], brecv.at[1]).wait_recv()
        for r in fwd3:
            r.wait_send()
        _recv(kb_ref.at[:, pl.ds(SKV_SHARD, SLIV)], svrecv.at[0]).wait_recv()
        _recv(vb_ref.at[:, pl.ds(SKV_SHARD, SLIV)], svrecv.at[1]).wait_recv()

    @pl.when(me == 2)
    def _():
        _recv(kb_ref.at[pl.ds(0, 4), pl.ds(0, SKV_SHARD)], frecv.at[0]).wait_recv()
        _recv(vb_ref.at[pl.ds(0, 4), pl.ds(0, SKV_SHARD)], frecv.at[1]).wait_recv()
        _recv(kb_ref.at[pl.ds(4, 4), pl.ds(0, SKV_SHARD)], frecv.at[2]).wait_recv()
        _recv(vb_ref.at[pl.ds(4, 4), pl.ds(0, SKV_SHARD)], frecv.at[3]).wait_recv()
        _recv(kb_ref.at[:, pl.ds(SKV_SHARD, SLIV)], svrecv.at[0]).wait_recv()
        _recv(vb_ref.at[:, pl.ds(SKV_SHARD, SLIV)], svrecv.at[1]).wait_recv()


def _exchange(kt, vt, q32):
    return pl.pallas_call(
        _exch_body,
        out_shape=[
            jax.ShapeDtypeStruct((HSH, BK, DH), jnp.bfloat16),
            jax.ShapeDtypeStruct((HSH, BK, DH), jnp.bfloat16),
            jax.ShapeDtypeStruct((HSH, NGLOB, DH), jnp.bfloat16),
        ],
        in_specs=[
            pl.BlockSpec(memory_space=pl.ANY),
            pl.BlockSpec(memory_space=pl.ANY),
            pl.BlockSpec(memory_space=pltpu.VMEM),
        ],
        out_specs=[
            pl.BlockSpec(memory_space=pl.ANY),
            pl.BlockSpec(memory_space=pl.ANY),
            pl.BlockSpec(memory_space=pltpu.VMEM),
        ],
        scratch_shapes=[
            pltpu.VMEM((NH, NGLOB, DH), jnp.bfloat16),
            pltpu.VMEM((2, HSH, SKV_SHARD, DH), jnp.bfloat16),
            pltpu.VMEM((2, HSH, SKV_SHARD, DH), jnp.bfloat16),
            pltpu.VMEM((N_DEV, HSH, NGLOB, DH), jnp.float32),
            pltpu.VMEM((N_DEV, 2, HSH, NGLOB), jnp.float32),
            pltpu.VMEM((N_DEV, HSH, NGLOB, DH), jnp.float32),
            pltpu.VMEM((N_DEV, 2, HSH, NGLOB), jnp.float32),
            pltpu.VMEM((4, SKV_SHARD, DH), jnp.bfloat16),
            pltpu.VMEM((4, SKV_SHARD, DH), jnp.bfloat16),
            pltpu.SemaphoreType.DMA((N_DEV,)),
            pltpu.SemaphoreType.DMA((N_DEV,)),
            pltpu.SemaphoreType.DMA((N_DEV,)),
            pltpu.SemaphoreType.DMA((N_DEV,)),
            pltpu.SemaphoreType.DMA((N_DEV,)),
            pltpu.SemaphoreType.DMA((N_DEV,)),
            pltpu.SemaphoreType.DMA((4,)),
            pltpu.SemaphoreType.DMA((2,)),
            pltpu.SemaphoreType.DMA((4,)),
            pltpu.SemaphoreType.DMA((2,)),
            pltpu.SemaphoreType.DMA((2,)),
            pltpu.SemaphoreType.DMA((4,)),
            pltpu.SemaphoreType.DMA((8,)),
            pltpu.SemaphoreType.DMA((2,)),
            pltpu.SemaphoreType.DMA((4,)),
            pltpu.SemaphoreType.DMA((2,)),
            pltpu.SemaphoreType.DMA((2,)),
        ],
    )(kt, vt, q32)


def _awa_body(q_ref, kb_ref, vb_ref, g32_ref, wo_ref, o_ref,
              rs_ref, s1send, s1recv, s2send, s2recv):
    me = lax.axis_index("i")
    c = pl.program_id(0)
    C = SQ // N_DEV
    lo = pl.multiple_of(jnp.maximum(c * C - BAND, 0), 128)
    qi0 = c * C

    ctxs = []
    for h in range(HSH):
        q = q_ref[h]
        kwin = kb_ref[h, pl.ds(lo, 768), :]
        vwin = vb_ref[h, pl.ds(lo, 768), :]

        sb = lax.dot_general(q, kwin, (((1,), (1,)), ((), ())),
                             preferred_element_type=jnp.float32) * SCALE
        qi = qi0 + lax.broadcasted_iota(jnp.int32, (C, 768), 0)
        ki = lo + lax.broadcasted_iota(jnp.int32, (C, 768), 1)
        mb = ((jnp.abs(qi - ki) <= BAND) | (ki < NGLOB)) & (qi >= NGLOB)
        sb = jnp.where(mb, sb, -1e9)

        sg = lax.dot_general(q, kb_ref[h, :128, :], (((1,), (1,)), ((), ())),
                             preferred_element_type=jnp.float32) * SCALE
        kig = lax.broadcasted_iota(jnp.int32, (C, 128), 1)
        mg = (kig < NGLOB) & (lo > 0)
        sg = jnp.where(mg, sg, -1e9)

        m = jnp.maximum(jnp.max(sb, axis=1, keepdims=True),
                        jnp.max(sg, axis=1, keepdims=True))
        wb = jnp.exp(sb - m)
        wg = jnp.exp(sg - m)
        l = jnp.sum(wb, axis=1, keepdims=True) + jnp.sum(wg, axis=1, keepdims=True)
        ctx = (lax.dot_general(wb.astype(jnp.bfloat16), vwin,
                               (((1,), (0,)), ((), ())),
                               preferred_element_type=jnp.float32)
               + lax.dot_general(wg.astype(jnp.bfloat16), vb_ref[h, :128, :],
                                 (((1,), (0,)), ((), ())),
                                 preferred_element_type=jnp.float32)) / l

        g32p = jnp.concatenate(
            [g32_ref[h].astype(jnp.float32),
             jnp.zeros((C - NGLOB, DH), jnp.float32)], axis=0)
        rows = lax.broadcasted_iota(jnp.int32, (C, 1), 0)
        ctx = jnp.where((rows < NGLOB) & (c == 0), g32p, ctx)
        ctxs.append(ctx.astype(jnp.bfloat16))

    ctx2 = jnp.concatenate(ctxs, axis=1)
    chunk = lax.dot_general(ctx2, wo_ref[...], (((1,), (0,)), ((), ())),
                            preferred_element_type=jnp.float32
                            ).astype(jnp.bfloat16)

    o_ref[pl.ds(c * C, C), :] = chunk
    for dd in (1, 2, 3):
        @pl.when(jnp.mod(c - me, N_DEV) == dd)
        def _():
            pltpu.make_async_remote_copy(
                src_ref=o_ref.at[pl.ds(c * C, C)],
                dst_ref=rs_ref.at[pl.ds((dd - 1) * C, C)],
                send_sem=s1send.at[dd], recv_sem=s1recv.at[dd],
                device_id=(jnp.mod(me + dd, N_DEV),),
                device_id_type=pl.DeviceIdType.MESH).start()

    @pl.when(c == N_DEV - 1)
    def _():
        acc = o_ref[pl.ds(me * C, C), :].astype(jnp.float32)
        for dd in (1, 2, 3):
            pltpu.make_async_remote_copy(
                src_ref=rs_ref.at[pl.ds((dd - 1) * C, C)],
                dst_ref=rs_ref.at[pl.ds((dd - 1) * C, C)],
                send_sem=s1send.at[dd], recv_sem=s1recv.at[dd],
                device_id=(0,),
                device_id_type=pl.DeviceIdType.MESH).wait_recv()
            acc = acc + rs_ref[(dd - 1) * C:dd * C, :].astype(jnp.float32)
        o_ref[pl.ds(me * C, C), :] = acc.astype(jnp.bfloat16)

        r2 = []
        for dd in (1, 2, 3):
            rd = pltpu.make_async_remote_copy(
                src_ref=o_ref.at[pl.ds(me * C, C)],
                dst_ref=o_ref.at[pl.ds(me * C, C)],
                send_sem=s2send.at[dd], recv_sem=s2recv.at[dd],
                device_id=(jnp.mod(me + dd, N_DEV),),
                device_id_type=pl.DeviceIdType.MESH)
            rd.start()
            r2.append(rd)
        for dd in (1, 2, 3):
            pltpu.make_async_remote_copy(
                src_ref=o_ref.at[pl.ds(0, C)],
                dst_ref=rs_ref.at[pl.ds(0, C)],
                send_sem=s1send.at[dd], recv_sem=s1recv.at[dd],
                device_id=(0,),
                device_id_type=pl.DeviceIdType.MESH).wait_send()
        for rd in r2:
            rd.wait()


def _attn_wo_ar(qh, kb, vb, g32, wo):
    return pl.pallas_call(
        _awa_body,
        grid=(N_DEV,),
        out_shape=jax.ShapeDtypeStruct((SQ, DM), jnp.bfloat16),
        in_specs=[
            pl.BlockSpec((HSH, SQ // N_DEV, DH), lambda c: (0, c, 0)),
            pl.BlockSpec((HSH, BK, DH), lambda c: (0, 0, 0)),
            pl.BlockSpec((HSH, BK, DH), lambda c: (0, 0, 0)),
            pl.BlockSpec((HSH, NGLOB, DH), lambda c: (0, 0, 0)),
            pl.BlockSpec((DM, DM), lambda c: (0, 0)),
        ],
        out_specs=pl.BlockSpec((SQ, DM), lambda c: (0, 0)),
        scratch_shapes=[
            pltpu.VMEM(((N_DEV - 1) * (SQ // N_DEV), DM), jnp.bfloat16),
            pltpu.SemaphoreType.DMA((N_DEV,)),
            pltpu.SemaphoreType.DMA((N_DEV,)),
            pltpu.SemaphoreType.DMA((N_DEV,)),
            pltpu.SemaphoreType.DMA((N_DEV,)),
        ],
    )(qh, kb, vb, g32, wo)


def kernel(x, Wq, K_ext, V_ext, Wo):
    xb = x[0].astype(jnp.bfloat16)
    q = xb @ Wq.astype(jnp.bfloat16)
    qh = q.reshape(SQ, HSH, DH).transpose(1, 0, 2)
    q32 = qh[:, :NGLOB, :]

    kt = K_ext[0].astype(jnp.bfloat16).transpose(1, 0, 2)
    vt = V_ext[0].astype(jnp.bfloat16).transpose(1, 0, 2)

    kb, vb, g32 = _exchange(kt, vt, q32)
    out = _attn_wo_ar(qh, kb, vb, g32, Wo.astype(jnp.bfloat16))
    return out.astype(jnp.float32).reshape(1, SQ, DM)


# device time: 274125 ns/iter; 1.5310x vs baseline; 1.0445x over previous
import jax
import jax.numpy as jnp
from jax import lax
from jax.experimental import pallas as pl
from jax.experimental.pallas import tpu as pltpu

N_DEV = 4
SQ = 2048
SKV_SHARD = 2048
HSH = 8
NH = 32
DH = 128
DM = 1024
BAND = 128
NGLOB = 32
SCALE = 0.08838834764831843
SLIV = 128
BK = SKV_SHARD + SLIV


def _exch_body(kt_ref, vt_ref, q32_ref,
               kb_ref, vb_ref, g32_ref,
               q32all, kc, vc, osend, stsend, oall, stall, rlk, rlv,
               qsend, qrecv, opsend, oprecv, stpsend, stprecv,
               bsend, brecv, rsend, rrecv, fsend, frecv,
               svsend, svrecv, lsem, cksem, cvsem):
    me = lax.axis_index("i")

    lq = pltpu.make_async_copy(q32_ref, q32all.at[pl.ds(me * HSH, HSH)],
                               lsem.at[2])
    lq.start()
    qr = []
    for d in (1, 2, 3):
        p = (me + d) % N_DEV
        r = pltpu.make_async_remote_copy(
            src_ref=q32_ref,
            dst_ref=q32all.at[pl.ds(me * HSH, HSH)],
            send_sem=qsend.at[d], recv_sem=qrecv.at[d],
            device_id=(p,), device_id_type=pl.DeviceIdType.MESH)
        r.start()
        qr.append(r)

    band0 = [
        pltpu.make_async_remote_copy(
            src_ref=kt_ref.at[pl.ds(16, 4)], dst_ref=rlk,
            send_sem=rsend.at[0], recv_sem=rrecv.at[0],
            device_id=(1,), device_id_type=pl.DeviceIdType.MESH),
        pltpu.make_async_remote_copy(
            src_ref=vt_ref.at[pl.ds(16, 4)], dst_ref=rlv,
            send_sem=rsend.at[1], recv_sem=rrecv.at[1],
            device_id=(1,), device_id_type=pl.DeviceIdType.MESH),
        pltpu.make_async_remote_copy(
            src_ref=kt_ref.at[pl.ds(20, 4)], dst_ref=rlk,
            send_sem=rsend.at[2], recv_sem=rrecv.at[0],
            device_id=(3,), device_id_type=pl.DeviceIdType.MESH),
        pltpu.make_async_remote_copy(
            src_ref=vt_ref.at[pl.ds(20, 4)], dst_ref=rlv,
            send_sem=rsend.at[3], recv_sem=rrecv.at[1],
            device_id=(3,), device_id_type=pl.DeviceIdType.MESH),
        pltpu.make_async_remote_copy(
            src_ref=kt_ref.at[pl.ds(8, 8)],
            dst_ref=kb_ref.at[:, pl.ds(0, SKV_SHARD)],
            send_sem=bsend.at[0], recv_sem=brecv.at[0],
            device_id=(1,), device_id_type=pl.DeviceIdType.MESH),
        pltpu.make_async_remote_copy(
            src_ref=vt_ref.at[pl.ds(8, 8)],
            dst_ref=vb_ref.at[:, pl.ds(0, SKV_SHARD)],
            send_sem=bsend.at[1], recv_sem=brecv.at[1],
            device_id=(1,), device_id_type=pl.DeviceIdType.MESH),
        pltpu.make_async_remote_copy(
            src_ref=kt_ref.at[pl.ds(24, 8)],
            dst_ref=kb_ref.at[:, pl.ds(0, SKV_SHARD)],
            send_sem=bsend.at[2], recv_sem=brecv.at[0],
            device_id=(3,), device_id_type=pl.DeviceIdType.MESH),
        pltpu.make_async_remote_copy(
            src_ref=vt_ref.at[pl.ds(24, 8)],
            dst_ref=vb_ref.at[:, pl.ds(0, SKV_SHARD)],
            send_sem=bsend.at[3], recv_sem=brecv.at[1],
            device_id=(3,), device_id_type=pl.DeviceIdType.MESH),
    ]
    loc0 = [
        pltpu.make_async_copy(kt_ref.at[pl.ds(0, HSH)],
                              kb_ref.at[:, pl.ds(0, SKV_SHARD)], lsem.at[0]),
        pltpu.make_async_copy(vt_ref.at[pl.ds(0, HSH)],
                              vb_ref.at[:, pl.ds(0, SKV_SHARD)], lsem.at[1]),
    ]

    sliv1 = []
    for i, p in enumerate((0, 2, 3)):
        sliv1.append(pltpu.make_async_remote_copy(
            src_ref=kt_ref.at[pl.ds(p * HSH, HSH), pl.ds(0, SLIV)],
            dst_ref=kb_ref.at[:, pl.ds(SKV_SHARD, SLIV)],
            send_sem=svsend.at[2 * i], recv_sem=svrecv.at[0],
            device_id=(p,), device_id_type=pl.DeviceIdType.MESH))
        sliv1.append(pltpu.make_async_remote_copy(
            src_ref=vt_ref.at[pl.ds(p * HSH, HSH), pl.ds(0, SLIV)],
            dst_ref=vb_ref.at[:, pl.ds(SKV_SHARD, SLIV)],
            send_sem=svsend.at[2 * i + 1], recv_sem=svrecv.at[1],
            device_id=(p,), device_id_type=pl.DeviceIdType.MESH))
    loc1 = [
        pltpu.make_async_copy(
            kt_ref.at[pl.ds(HSH, HSH), pl.ds(0, SLIV)],
            kb_ref.at[:, pl.ds(SKV_SHARD, SLIV)], lsem.at[0]),
        pltpu.make_async_copy(
            vt_ref.at[pl.ds(HSH, HSH), pl.ds(0, SLIV)],
            vb_ref.at[:, pl.ds(SKV_SHARD, SLIV)], lsem.at[1]),
    ]

    relay_recv = [
        pltpu.make_async_remote_copy(
            src_ref=rlk, dst_ref=rlk, send_sem=lsem.at[3],
            recv_sem=rrecv.at[0], device_id=(0,),
            device_id_type=pl.DeviceIdType.MESH),
        pltpu.make_async_remote_copy(
            src_ref=rlv, dst_ref=rlv, send_sem=lsem.at[3],
            recv_sem=rrecv.at[1], device_id=(0,),
            device_id_type=pl.DeviceIdType.MESH),
    ]
    fwd1 = [
        pltpu.make_async_remote_copy(
            src_ref=rlk, dst_ref=kb_ref.at[pl.ds(0, 4), pl.ds(0, SKV_SHARD)],
            send_sem=fsend.at[0], recv_sem=frecv.at[0],
            device_id=(2,), device_id_type=pl.DeviceIdType.MESH),
        pltpu.make_async_remote_copy(
            src_ref=rlv, dst_ref=vb_ref.at[pl.ds(0, 4), pl.ds(0, SKV_SHARD)],
            send_sem=fsend.at[1], recv_sem=frecv.at[1],
            device_id=(2,), device_id_type=pl.DeviceIdType.MESH),
    ]
    fwd3 = [
        pltpu.make_async_remote_copy(
            src_ref=rlk, dst_ref=kb_ref.at[pl.ds(4, 4), pl.ds(0, SKV_SHARD)],
            send_sem=fsend.at[0], recv_sem=frecv.at[2],
            device_id=(2,), device_id_type=pl.DeviceIdType.MESH),
        pltpu.make_async_remote_copy(
            src_ref=rlv, dst_ref=vb_ref.at[pl.ds(4, 4), pl.ds(0, SKV_SHARD)],
            send_sem=fsend.at[1], recv_sem=frecv.at[3],
            device_id=(2,), device_id_type=pl.DeviceIdType.MESH),
    ]

    @pl.when(me == 0)
    def _():
        for r in band0:
            r.start()
        for c in loc0:
            c.start()

    @pl.when(me == 1)
    def _():
        for r in sliv1:
            r.start()
        for c in loc1:
            c.start()

    @pl.when(me == 1)
    def _():
        relay_recv[0].wait_recv()
        relay_recv[1].wait_recv()
        for r in fwd1:
            r.start()

    @pl.when(me == 3)
    def _():
        relay_recv[0].wait_recv()
        relay_recv[1].wait_recv()
        for r in fwd3:
            r.start()

    for r in qr:
        r.wait()
    lq.wait()
    qv = q32all[...]

    ck0 = pltpu.make_async_copy(kt_ref.at[pl.ds(0, HSH)], kc.at[0], cksem.at[0])
    cv0 = pltpu.make_async_copy(vt_ref.at[pl.ds(0, HSH)], vc.at[0], cvsem.at[0])
    ck0.start()
    cv0.start()
    copies = [(ck0, cv0)]
    for hc in range(N_DEV):
        if hc + 1 < N_DEV:
            nk = pltpu.make_async_copy(
                kt_ref.at[pl.ds((hc + 1) * HSH, HSH)], kc.at[(hc + 1) % 2],
                cksem.at[(hc + 1) % 2])
            nv = pltpu.make_async_copy(
                vt_ref.at[pl.ds((hc + 1) * HSH, HSH)], vc.at[(hc + 1) % 2],
                cvsem.at[(hc + 1) % 2])
            nk.start()
            nv.start()
            copies.append((nk, nv))
        copies[hc][0].wait()
        copies[hc][1].wait()
        qc = qv[hc * HSH:(hc + 1) * HSH]
        s = lax.dot_general(qc, kc[hc % 2], (((2,), (2,)), ((0,), (0,))),
                            preferred_element_type=jnp.float32) * SCALE
        m_c = jnp.max(s, axis=2)
        w = jnp.exp(s - m_c[:, :, None])
        l_c = jnp.sum(w, axis=2)
        o_c = lax.dot_general(w.astype(jnp.bfloat16), vc[hc % 2],
                              (((2,), (1,)), ((0,), (0,))),
                              preferred_element_type=jnp.float32)
        osend[hc] = o_c
        stsend[hc, 0] = m_c
        stsend[hc, 1] = l_c

    oall[pl.ds(me, 1)] = osend[pl.ds(me, 1)]
    stall[pl.ds(me, 1)] = stsend[pl.ds(me, 1)]
    pr = []
    for d in (1, 2, 3):
        p = (me + d) % N_DEV
        r1 = pltpu.make_async_remote_copy(
            src_ref=osend.at[pl.ds(p, 1)], dst_ref=oall.at[pl.ds(me, 1)],
            send_sem=opsend.at[d], recv_sem=oprecv.at[d],
            device_id=(p,), device_id_type=pl.DeviceIdType.MESH)
        r2 = pltpu.make_async_remote_copy(
            src_ref=stsend.at[pl.ds(p, 1)], dst_ref=stall.at[pl.ds(me, 1)],
            send_sem=stpsend.at[d], recv_sem=stprecv.at[d],
            device_id=(p,), device_id_type=pl.DeviceIdType.MESH)
        r1.start()
        r2.start()
        pr.append((r1, r2))
    for r1, r2 in pr:
        r1.wait()
        r2.wait()

    ov = oall[...]
    stv = stall[...]
    mj = stv[:, 0]
    mm = jnp.max(mj, axis=0)
    a = jnp.exp(mj - mm[None])
    ctx = jnp.sum(a[..., None] * ov, axis=0)
    ll = jnp.sum(a * stv[:, 1], axis=0)
    g32_ref[...] = (ctx / ll[..., None]).astype(jnp.bfloat16)

    def _recv(dst, sem):
        return pltpu.make_async_remote_copy(
            src_ref=dst, dst_ref=dst, send_sem=lsem.at[3], recv_sem=sem,
            device_id=(0,), device_id_type=pl.DeviceIdType.MESH)

    @pl.when(me == 0)
    def _():
        _recv(kb_ref.at[:, pl.ds(SKV_SHARD, SLIV)], svrecv.at[0]).wait_recv()
        _recv(vb_ref.at[:, pl.ds(SKV_SHARD, SLIV)], svrecv.at[1]).wait_recv()
        for r in band0:
            r.wait_send()
        for c in loc0:
            c.wait()

    @pl.when(me == 1)
    def _():
        _recv(kb_ref.at[:, pl.ds(0, SKV_SHARD)], brecv.at[0]).wait_recv()
        _recv(vb_ref.at[:, pl.ds(0, SKV_SHARD)], brecv.at[1]).wait_recv()
        for r in fwd1:
            r.wait_send()
        for r in sliv1:
            r.wait_send()
        for c in loc1:
            c.wait()

    @pl.when(me == 3)
    def _():
        _recv(kb_ref.at[:, pl.ds(0, SKV_SHARD)], brecv.at[0]).wait_recv()
        _recv(vb_ref.at[:, pl.ds(0, SKV_SHARD)], brecv.at[1]).wait_recv()
        for r in fwd3:
            r.wait_send()
        _recv(kb_ref.at[:, pl.ds(SKV_SHARD, SLIV)], svrecv.at[0]).wait_recv()
        _recv(vb_ref.at[:, pl.ds(SKV_SHARD, SLIV)], svrecv.at[1]).wait_recv()

    @pl.when(me == 2)
    def _():
        _recv(kb_ref.at[pl.ds(0, 4), pl.ds(0, SKV_SHARD)], frecv.at[0]).wait_recv()
        _recv(vb_ref.at[pl.ds(0, 4), pl.ds(0, SKV_SHARD)], frecv.at[1]).wait_recv()
        _recv(kb_ref.at[pl.ds(4, 4), pl.ds(0, SKV_SHARD)], frecv.at[2]).wait_recv()
        _recv(vb_ref.at[pl.ds(4, 4), pl.ds(0, SKV_SHARD)], frecv.at[3]).wait_recv()
        _recv(kb_ref.at[:, pl.ds(SKV_SHARD, SLIV)], svrecv.at[0]).wait_recv()
        _recv(vb_ref.at[:, pl.ds(SKV_SHARD, SLIV)], svrecv.at[1]).wait_recv()


def _exchange(kt, vt, q32):
    return pl.pallas_call(
        _exch_body,
        out_shape=[
            jax.ShapeDtypeStruct((HSH, BK, DH), jnp.bfloat16),
            jax.ShapeDtypeStruct((HSH, BK, DH), jnp.bfloat16),
            jax.ShapeDtypeStruct((HSH, NGLOB, DH), jnp.bfloat16),
        ],
        in_specs=[
            pl.BlockSpec(memory_space=pl.ANY),
            pl.BlockSpec(memory_space=pl.ANY),
            pl.BlockSpec(memory_space=pltpu.VMEM),
        ],
        out_specs=[
            pl.BlockSpec(memory_space=pl.ANY),
            pl.BlockSpec(memory_space=pl.ANY),
            pl.BlockSpec(memory_space=pltpu.VMEM),
        ],
        scratch_shapes=[
            pltpu.VMEM((NH, NGLOB, DH), jnp.bfloat16),
            pltpu.VMEM((2, HSH, SKV_SHARD, DH), jnp.bfloat16),
            pltpu.VMEM((2, HSH, SKV_SHARD, DH), jnp.bfloat16),
            pltpu.VMEM((N_DEV, HSH, NGLOB, DH), jnp.float32),
            pltpu.VMEM((N_DEV, 2, HSH, NGLOB), jnp.float32),
            pltpu.VMEM((N_DEV, HSH, NGLOB, DH), jnp.float32),
            pltpu.VMEM((N_DEV, 2, HSH, NGLOB), jnp.float32),
            pltpu.VMEM((4, SKV_SHARD, DH), jnp.bfloat16),
            pltpu.VMEM((4, SKV_SHARD, DH), jnp.bfloat16),
            pltpu.SemaphoreType.DMA((N_DEV,)),
            pltpu.SemaphoreType.DMA((N_DEV,)),
            pltpu.SemaphoreType.DMA((N_DEV,)),
            pltpu.SemaphoreType.DMA((N_DEV,)),
            pltpu.SemaphoreType.DMA((N_DEV,)),
            pltpu.SemaphoreType.DMA((N_DEV,)),
            pltpu.SemaphoreType.DMA((4,)),
            pltpu.SemaphoreType.DMA((2,)),
            pltpu.SemaphoreType.DMA((4,)),
            pltpu.SemaphoreType.DMA((2,)),
            pltpu.SemaphoreType.DMA((2,)),
            pltpu.SemaphoreType.DMA((4,)),
            pltpu.SemaphoreType.DMA((8,)),
            pltpu.SemaphoreType.DMA((2,)),
            pltpu.SemaphoreType.DMA((4,)),
            pltpu.SemaphoreType.DMA((2,)),
            pltpu.SemaphoreType.DMA((2,)),
        ],
    )(kt, vt, q32)


def _awa_body(q_ref, kb_ref, vb_ref, g32_ref, wo_ref, o_ref,
              rs_ref, s1send, s1recv, s2send, s2recv):
    me = lax.axis_index("i")
    step = pl.program_id(0)
    c = jnp.mod(me + 1 + step, N_DEV)
    C = SQ // N_DEV
    lo = pl.multiple_of(jnp.maximum(c * C - BAND, 0), 128)
    qi0 = c * C

    ctxs = []
    for h in range(HSH):
        q = q_ref[h]
        kwin = kb_ref[h, pl.ds(lo, 768), :]
        vwin = vb_ref[h, pl.ds(lo, 768), :]

        sb = lax.dot_general(q, kwin, (((1,), (1,)), ((), ())),
                             preferred_element_type=jnp.float32) * SCALE
        qi = qi0 + lax.broadcasted_iota(jnp.int32, (C, 768), 0)
        ki = lo + lax.broadcasted_iota(jnp.int32, (C, 768), 1)
        mb = ((jnp.abs(qi - ki) <= BAND) | (ki < NGLOB)) & (qi >= NGLOB)
        sb = jnp.where(mb, sb, -1e9)

        sg = lax.dot_general(q, kb_ref[h, :128, :], (((1,), (1,)), ((), ())),
                             preferred_element_type=jnp.float32) * SCALE
        kig = lax.broadcasted_iota(jnp.int32, (C, 128), 1)
        mg = (kig < NGLOB) & (lo > 0)
        sg = jnp.where(mg, sg, -1e9)

        m = jnp.maximum(jnp.max(sb, axis=1, keepdims=True),
                        jnp.max(sg, axis=1, keepdims=True))
        wb = jnp.exp(sb - m)
        wg = jnp.exp(sg - m)
        l = jnp.sum(wb, axis=1, keepdims=True) + jnp.sum(wg, axis=1, keepdims=True)
        ctx = (lax.dot_general(wb.astype(jnp.bfloat16), vwin,
                               (((1,), (0,)), ((), ())),
                               preferred_element_type=jnp.float32)
               + lax.dot_general(wg.astype(jnp.bfloat16), vb_ref[h, :128, :],
                                 (((1,), (0,)), ((), ())),
                                 preferred_element_type=jnp.float32)) / l

        g32p = jnp.concatenate(
            [g32_ref[h].astype(jnp.float32),
             jnp.zeros((C - NGLOB, DH), jnp.float32)], axis=0)
        rows = lax.broadcasted_iota(jnp.int32, (C, 1), 0)
        ctx = jnp.where((rows < NGLOB) & (c == 0), g32p, ctx)
        ctxs.append(ctx.astype(jnp.bfloat16))

    ctx2 = jnp.concatenate(ctxs, axis=1)
    chunk = lax.dot_general(ctx2, wo_ref[...], (((1,), (0,)), ((), ())),
                            preferred_element_type=jnp.float32
                            ).astype(jnp.bfloat16)

    row0 = pl.multiple_of(c * C, C)
    o_ref[pl.ds(row0, C), :] = chunk
    for dd in (1, 2, 3):
        @pl.when(step == dd - 1)
        def _():
            pltpu.make_async_remote_copy(
                src_ref=o_ref.at[pl.ds(row0, C)],
                dst_ref=rs_ref.at[pl.ds((dd - 1) * C, C)],
                send_sem=s1send.at[dd], recv_sem=s1recv.at[dd],
                device_id=(jnp.mod(me + dd, N_DEV),),
                device_id_type=pl.DeviceIdType.MESH).start()

    @pl.when(step == N_DEV - 1)
    def _():
        acc = o_ref[pl.ds(me * C, C), :].astype(jnp.float32)
        for dd in (1, 2, 3):
            pltpu.make_async_remote_copy(
                src_ref=rs_ref.at[pl.ds((dd - 1) * C, C)],
                dst_ref=rs_ref.at[pl.ds((dd - 1) * C, C)],
                send_sem=s1send.at[dd], recv_sem=s1recv.at[dd],
                device_id=(0,),
                device_id_type=pl.DeviceIdType.MESH).wait_recv()
            acc = acc + rs_ref[(dd - 1) * C:dd * C, :].astype(jnp.float32)
        o_ref[pl.ds(me * C, C), :] = acc.astype(jnp.bfloat16)

        r2 = []
        for dd in (1, 2, 3):
            rd = pltpu.make_async_remote_copy(
                src_ref=o_ref.at[pl.ds(me * C, C)],
                dst_ref=o_ref.at[pl.ds(me * C, C)],
                send_sem=s2send.at[dd], recv_sem=s2recv.at[dd],
                device_id=(jnp.mod(me + dd, N_DEV),),
                device_id_type=pl.DeviceIdType.MESH)
            rd.start()
            r2.append(rd)
        for dd in (1, 2, 3):
            pltpu.make_async_remote_copy(
                src_ref=o_ref.at[pl.ds(0, C)],
                dst_ref=rs_ref.at[pl.ds(0, C)],
                send_sem=s1send.at[dd], recv_sem=s1recv.at[dd],
                device_id=(0,),
                device_id_type=pl.DeviceIdType.MESH).wait_send()
        for rd in r2:
            rd.wait()


def _attn_wo_ar(qh, kb, vb, g32, wo):
    return pl.pallas_call(
        _awa_body,
        grid=(N_DEV,),
        out_shape=jax.ShapeDtypeStruct((SQ, DM), jnp.bfloat16),
        in_specs=[
            pl.BlockSpec((HSH, SQ // N_DEV, DH), lambda c: (0, c, 0)),
            pl.BlockSpec((HSH, BK, DH), lambda c: (0, 0, 0)),
            pl.BlockSpec((HSH, BK, DH), lambda c: (0, 0, 0)),
            pl.BlockSpec((HSH, NGLOB, DH), lambda c: (0, 0, 0)),
            pl.BlockSpec((DM, DM), lambda c: (0, 0)),
        ],
        out_specs=pl.BlockSpec((SQ, DM), lambda c: (0, 0)),
        scratch_shapes=[
            pltpu.VMEM(((N_DEV - 1) * (SQ // N_DEV), DM), jnp.bfloat16),
            pltpu.SemaphoreType.DMA((N_DEV,)),
            pltpu.SemaphoreType.DMA((N_DEV,)),
            pltpu.SemaphoreType.DMA((N_DEV,)),
            pltpu.SemaphoreType.DMA((N_DEV,)),
        ],
    )(qh, kb, vb, g32, wo)


def kernel(x, Wq, K_ext, V_ext, Wo):
    xb = x[0].astype(jnp.bfloat16)
    q = xb @ Wq.astype(jnp.bfloat16)
    qh = q.reshape(SQ, HSH, DH).transpose(1, 0, 2)
    q32 = qh[:, :NGLOB, :]

    kt = K_ext[0].astype(jnp.bfloat16).transpose(1, 0, 2)
    vt = V_ext[0].astype(jnp.bfloat16).transpose(1, 0, 2)

    kb, vb, g32 = _exchange(kt, vt, q32)
    me = lax.axis_index("i")
    qh_rot = jnp.roll(qh, -(me + 1) * (SQ // N_DEV), axis=1)
    out = _attn_wo_ar(qh_rot, kb, vb, g32, Wo.astype(jnp.bfloat16))
    return out.astype(jnp.float32).reshape(1, SQ, DM)
